# baseline (speedup 1.0000x reference)
import functools

import jax
import jax.numpy as jnp
from jax import lax
from jax.experimental import pallas as pl
from jax.experimental.pallas import tpu as pltpu
from jax.experimental.pallas import tpu_sc as plsc

N = 10000
E = 320000
H = 128
HW = H // 2
NC = 2
NS = 16
NW = NC * NS
CH = 128

_mesh = plsc.VectorSubcoreMesh(core_axis_name="c", subcore_axis_name="s")

f32 = jnp.float32
bf16 = jnp.bfloat16
u16 = jnp.uint16
u32 = jnp.uint32
i32 = jnp.int32


def _silu(v):
    return v * (1.0 / (1.0 + jnp.exp(-v)))


@functools.partial(
    pl.kernel,
    out_type=jax.ShapeDtypeStruct((E,), f32),
    mesh=_mesh,
    compiler_params=pltpu.CompilerParams(needs_layout_passes=False),
    scratch_types=[
        pltpu.VMEM((4, N), f32),
        pltpu.VMEM((CH,), i32),
        pltpu.VMEM((CH,), i32),
        pltpu.VMEM((CH,), f32),
    ],
)
def _radial_sc(xt_hbm, row_hbm, col_hbm, rad_hbm, xt_v, row_v, col_v, rad_v):
    cid = lax.axis_index("c")
    sid = lax.axis_index("s")
    wid = sid * NC + cid
    pltpu.sync_copy(xt_hbm, xt_v)
    nchunk = E // CH

    def body(j, carry):
        chunk = wid + j * NW

        @pl.when(chunk < nchunk)
        def _():
            base = chunk * CH
            pltpu.sync_copy(row_hbm.at[pl.ds(base, CH)], row_v)
            pltpu.sync_copy(col_hbm.at[pl.ds(base, CH)], col_v)

            def sub(k, c2):
                ridx = row_v[pl.ds(k * 16, 16)]
                cidx = col_v[pl.ds(k * 16, 16)]
                acc = jnp.zeros((16,), f32)
                for d in range(3):
                    didx = jnp.full((16,), d, i32)
                    xr = plsc.load_gather(xt_v, [didx, ridx])
                    xc = plsc.load_gather(xt_v, [didx, cidx])
                    dd = xr - xc
                    acc = acc + dd * dd
                rad_v[pl.ds(k * 16, 16)] = acc
                return c2

            lax.fori_loop(0, CH // 16, sub, 0)
            pltpu.sync_copy(rad_v, rad_hbm.at[pl.ds(base, CH)])

        return carry

    lax.fori_loop(0, (nchunk + NW - 1) // NW, body, 0)


NCHUNK = E // CH
PER_W = NCHUNK // NW
TAILC = NCHUNK - PER_W * NW
NITER = PER_W // 2


def _sum_pack_rows(a_v, b_v, s_v):
    def rowbody(r, c):
        for g in range(H // 32):
            lo = pl.ds(g * 16, 16)
            hi = pl.ds(64 + g * 16, 16)
            x = a_v[r, lo] + b_v[r, lo]
            y = a_v[r, hi] + b_v[r, hi]
            p = plsc.pack(x, y, format=plsc.PackFormat.INTERLEAVED)
            s_v[r, pl.ds(g * 16, 16)] = plsc.bitcast(p, u32)
        return c

    lax.fori_loop(0, CH, rowbody, 0)


@functools.partial(
    pl.kernel,
    out_type=jax.ShapeDtypeStruct((E, HW), u32),
    mesh=_mesh,
    compiler_params=pltpu.CompilerParams(needs_layout_passes=False),
    scratch_types=[
        pltpu.VMEM((CH,), i32), pltpu.VMEM((CH,), i32),
        pltpu.VMEM((CH,), i32), pltpu.VMEM((CH,), i32),
        pltpu.VMEM((CH, H), f32), pltpu.VMEM((CH, H), f32),
        pltpu.VMEM((CH, H), f32), pltpu.VMEM((CH, H), f32),
        pltpu.VMEM((CH, HW), u32), pltpu.VMEM((CH, HW), u32),
        pltpu.SemaphoreType.DMA, pltpu.SemaphoreType.DMA,
        pltpu.SemaphoreType.DMA, pltpu.SemaphoreType.DMA,
        pltpu.SemaphoreType.DMA, pltpu.SemaphoreType.DMA,
    ],
)
def _gather_add_sc(a_hbm, b_hbm, row_hbm, col_hbm, s_hbm,
                   rv0, rv1, cv0, cv1, av0, av1, bv0, bv1, sv0, sv1,
                   si0, si1, sg0, sg1, sw0, sw1):
    cid = lax.axis_index("c")
    sid = lax.axis_index("s")
    wid = sid * NC + cid
    start = wid * PER_W
    rv = (rv0, rv1)
    cv = (cv0, cv1)
    av = (av0, av1)
    bv = (bv0, bv1)
    sv = (sv0, sv1)
    si = (si0, si1)
    sg = (sg0, sg1)
    sw = (sw0, sw1)

    def issue_idx(t, p):
        base = t * CH
        pltpu.async_copy(row_hbm.at[pl.ds(base, CH)], rv[p], si[p])
        pltpu.async_copy(col_hbm.at[pl.ds(base, CH)], cv[p], si[p])

    def wait_idx(p):
        pltpu.make_async_copy(row_hbm.at[pl.ds(0, CH)], rv[p], si[p]).wait()
        pltpu.make_async_copy(col_hbm.at[pl.ds(0, CH)], cv[p], si[p]).wait()

    def issue_gather(p):
        pltpu.async_copy(a_hbm.at[rv[p]], av[p], sg[p])
        pltpu.async_copy(b_hbm.at[cv[p]], bv[p], sg[p])

    def wait_gather(p):
        pltpu.make_async_copy(a_hbm.at[pl.ds(0, CH)], av[p], sg[p]).wait()
        pltpu.make_async_copy(b_hbm.at[pl.ds(0, CH)], bv[p], sg[p]).wait()

    def issue_write(t, p):
        pltpu.async_copy(sv[p], s_hbm.at[pl.ds(t * CH, CH)], sw[p])

    def wait_write(p):
        pltpu.make_async_copy(s_hbm.at[pl.ds(0, CH)], sv[p], sw[p]).wait()

    issue_idx(start, 0)
    wait_idx(0)
    issue_gather(0)

    def body(jj, carry):
        t0 = start + 2 * jj
        t1 = t0 + 1

        @pl.when(jj > 0)
        def _():
            wait_write(1)

        issue_idx(t1, 1)
        wait_idx(1)
        issue_gather(1)

        wait_gather(0)
        _sum_pack_rows(av0, bv0, sv0)
        issue_write(t0, 0)

        @pl.when(jj + 1 < NITER)
        def _():
            wait_write(0)
            issue_idx(t0 + 2, 0)
            wait_idx(0)
            issue_gather(0)

        wait_gather(1)
        _sum_pack_rows(av1, bv1, sv1)
        issue_write(t1, 1)
        return carry

    lax.fori_loop(0, NITER, body, 0)
    wait_write(0)
    wait_write(1)

    @pl.when(wid < TAILC)
    def _():
        t = NW * PER_W + wid
        issue_idx(t, 0)
        wait_idx(0)
        issue_gather(0)
        wait_gather(0)
        _sum_pack_rows(av0, bv0, sv0)
        issue_write(t, 0)
        wait_write(0)


NP = 10112
CHS = 64
SC_CHUNKS = (E // NC) // CHS
SC_PER_T = SC_CHUNKS // NS
SC_TAIL = SC_CHUNKS - SC_PER_T * NS


def _unpack_rows(m_v, f_v):
    def rowbody(r, c):
        for g in range(H // 32):
            w = m_v[r, pl.ds(g * 16, 16)]
            p = plsc.bitcast(w, bf16)
            x, y = plsc.unpack(p, format=plsc.PackFormat.INTERLEAVED)
            f_v[r, pl.ds(g * 16, 16)] = x
            f_v[r, pl.ds(64 + g * 16, 16)] = y
        return c

    lax.fori_loop(0, CHS, rowbody, 0)


@functools.partial(
    pl.kernel,
    out_type=jax.ShapeDtypeStruct((NC, NP, H), f32),
    mesh=_mesh,
    compiler_params=pltpu.CompilerParams(needs_layout_passes=False),
    scratch_types=[
        pltpu.VMEM_SHARED((NP, H), f32),
        pltpu.VMEM((CHS,), i32), pltpu.VMEM((CHS,), i32),
        pltpu.VMEM((CHS, HW), u32), pltpu.VMEM((CHS, HW), u32),
        pltpu.VMEM((CHS, H), f32), pltpu.VMEM((CHS, H), f32),
        pltpu.SemaphoreType.DMA, pltpu.SemaphoreType.DMA,
        pltpu.SemaphoreType.DMA, pltpu.SemaphoreType.DMA,
    ],
)
def _scatter_sc(m_hbm, row_hbm, zero_hbm, out_hbm, acc_sh,
                iv0, iv1, mv0, mv1, fv0, fv1, sl0, sl1, ss0, ss1):
    cid = lax.axis_index("c")
    sid = lax.axis_index("s")
    rows_per_tile = NP // NS
    r0 = sid * rows_per_tile
    pltpu.sync_copy(zero_hbm.at[pl.ds(r0, rows_per_tile)],
                    acc_sh.at[pl.ds(r0, rows_per_tile)])
    plsc.subcore_barrier()

    iv = (iv0, iv1)
    mv = (mv0, mv1)
    fv = (fv0, fv1)
    sl = (sl0, sl1)
    ss = (ss0, ss1)
    start = cid * SC_CHUNKS + sid * SC_PER_T

    def issue_load(t, p):
        base = t * CHS
        pltpu.async_copy(row_hbm.at[pl.ds(base, CHS)], iv[p], sl[p])
        pltpu.async_copy(m_hbm.at[pl.ds(base, CHS)], mv[p], sl[p])

    def wait_load(p):
        pltpu.make_async_copy(row_hbm.at[pl.ds(0, CHS)], iv[p], sl[p]).wait()
        pltpu.make_async_copy(m_hbm.at[pl.ds(0, CHS)], mv[p], sl[p]).wait()

    def issue_scat(p):
        pltpu.async_copy(fv[p], acc_sh.at[iv[p]], ss[p], add=True)

    def wait_scat(p):
        pltpu.make_async_copy(zero_hbm.at[pl.ds(0, CHS)], fv[p], ss[p]).wait()

    issue_load(start, 0)

    def body(jj, carry):
        t0 = start + 2 * jj
        t1 = t0 + 1

        @pl.when(jj > 0)
        def _():
            wait_scat(1)

        issue_load(t1, 1)
        wait_load(0)
        _unpack_rows(mv0, fv0)
        issue_scat(0)

        @pl.when(jj + 1 < SC_PER_T // 2)
        def _():
            wait_scat(0)
            issue_load(t0 + 2, 0)

        wait_load(1)
        _unpack_rows(mv1, fv1)
        issue_scat(1)
        return carry

    lax.fori_loop(0, SC_PER_T // 2, body, 0)
    wait_scat(0)
    wait_scat(1)

    @pl.when(sid < SC_TAIL)
    def _():
        t = cid * SC_CHUNKS + NS * SC_PER_T + sid
        issue_load(t, 0)
        wait_load(0)
        _unpack_rows(mv0, fv0)
        issue_scat(0)
        wait_scat(0)

    plsc.subcore_barrier()
    pltpu.sync_copy(acc_sh.at[pl.ds(r0, rows_per_tile)],
                    out_hbm.at[cid, pl.ds(r0, rows_per_tile)])


BN = 1000
BE = 2000


def _bdot(x, w):
    return jnp.dot(x.astype(bf16), w, preferred_element_type=f32)


def _unpack_tc(u):
    lo = jax.lax.bitcast_convert_type((u & 0xFFFF).astype(u16), bf16)
    hi = jax.lax.bitcast_convert_type((u >> 16).astype(u16), bf16)
    return jnp.concatenate([lo.astype(f32), hi.astype(f32)], axis=1)


def _pack_tc(v):
    v16 = v.astype(bf16)
    lo = jax.lax.bitcast_convert_type(v16[:, :HW], u16).astype(u32)
    hi = jax.lax.bitcast_convert_type(v16[:, HW:], u16).astype(u32)
    return lo | (hi << 16)


def _node_first_body(h_ref, wemb_ref, bemb_ref, w1s_ref, w1d_ref,
                     hh_ref, a_ref, b_ref):
    hh = _bdot(h_ref[...], wemb_ref[...]) + bemb_ref[...]
    hh_ref[...] = hh
    a_ref[...] = _bdot(hh, w1s_ref[...])
    b_ref[...] = _bdot(hh, w1d_ref[...])


def _node_mid_body(hh_ref, agg_ref, nw1a_ref, nw1b_ref, nb1_ref,
                   nw2_ref, nb2_ref, w1s_ref, w1d_ref,
                   hh_out_ref, a_ref, b_ref):
    hh = hh_ref[...]
    agg = agg_ref[0] + agg_ref[1]
    u = _silu(_bdot(hh, nw1a_ref[...]) + _bdot(agg, nw1b_ref[...])
              + nb1_ref[...])
    out = _bdot(u, nw2_ref[...]) + nb2_ref[...]
    hh = hh + out
    hh_out_ref[...] = hh
    a_ref[...] = _bdot(hh, w1s_ref[...])
    b_ref[...] = _bdot(hh, w1d_ref[...])


def _node_last_body(hh_ref, agg_ref, nw1a_ref, nw1b_ref, nb1_ref,
                    nw2_ref, nb2_ref, wout_ref, bout_ref, hout_ref):
    hh = hh_ref[...]
    agg = agg_ref[0] + agg_ref[1]
    u = _silu(_bdot(hh, nw1a_ref[...]) + _bdot(agg, nw1b_ref[...])
              + nb1_ref[...])
    out = _bdot(u, nw2_ref[...]) + nb2_ref[...]
    hh = hh + out
    hout_ref[...] = _bdot(hh, wout_ref[...]) + bout_ref[...]


def _edge_body(s_ref, rad_ref, ea_ref, w1rad_ref, w1e_ref,
               b1_ref, w2_ref, b2_ref, m_ref):
    pre = (_unpack_tc(s_ref[...])
           + rad_ref[...] * w1rad_ref[...]
           + _bdot(ea_ref[...], w1e_ref[...])
           + b1_ref[...])
    t = _silu(pre)
    mm = jnp.dot(t.astype(bf16), w2_ref[...],
                 preferred_element_type=f32) + b2_ref[...]
    m_ref[...] = _pack_tc(_silu(mm))


def _wspec(shape):
    return pl.BlockSpec(shape, lambda i: tuple(0 for _ in shape))


def _node_call(body, out_dtypes, extra_in_specs):
    grid = N // BN
    rowspec = pl.BlockSpec((BN, H), lambda i: (i, 0))
    in_specs = [rowspec] + extra_in_specs
    out_specs = [rowspec] * len(out_dtypes)
    out_shape = [jax.ShapeDtypeStruct((N, H), dt) for dt in out_dtypes]
    if len(out_dtypes) == 1:
        out_specs = out_specs[0]
        out_shape = out_shape[0]
    return pl.pallas_call(body, grid=grid, in_specs=in_specs,
                          out_specs=out_specs, out_shape=out_shape)


_W = _wspec((H, H))
_BIAS = _wspec((1, H))
_AGGSPEC = pl.BlockSpec((NC, BN, H), lambda i: (0, i, 0))

_node_first = _node_call(_node_first_body, [f32, f32, f32],
                         [_W, _BIAS, _W, _W])
_node_mid = _node_call(_node_mid_body, [f32, f32, f32],
                       [_AGGSPEC, _W, _W, _BIAS, _W, _BIAS, _W, _W])
_node_last = _node_call(_node_last_body, [f32],
                        [_AGGSPEC, _W, _W, _BIAS, _W, _BIAS, _W, _BIAS])

_edge_mlp = pl.pallas_call(
    _edge_body,
    grid=E // BE,
    in_specs=[
        pl.BlockSpec((BE, HW), lambda i: (i, 0)),
        pl.BlockSpec((BE, 1), lambda i: (i, 0)),
        pl.BlockSpec((BE, 16), lambda i: (i, 0)),
        _BIAS,
        _wspec((16, H)),
        _BIAS,
        _W,
        _BIAS,
    ],
    out_specs=pl.BlockSpec((BE, HW), lambda i: (i, 0)),
    out_shape=jax.ShapeDtypeStruct((E, HW), u32),
)


def kernel(h, x, edge_attr, emb_in_w, emb_in_b, edge_w1, edge_b1, edge_w2,
           edge_b2, node_w1, node_b1, node_w2, node_b2, emb_out_w, emb_out_b,
           edge_index):
    L = edge_w1.shape[0]
    row = edge_index[0]
    col = edge_index[1]

    xt = jnp.zeros((4, N), f32).at[:3].set(x.T)
    radial = _radial_sc(xt, row, col).reshape(E, 1)

    zeros_nh = jnp.zeros((NP, H), f32)

    w1s = [edge_w1[i, :H] for i in range(L)]
    w1d = [edge_w1[i, H:2 * H] for i in range(L)]
    w1rad = [edge_w1[i, 2 * H].reshape(1, H) for i in range(L)]
    w1e = [edge_w1[i, 2 * H + 1:].astype(bf16) for i in range(L)]
    eb1 = [edge_b1[i].reshape(1, H) for i in range(L)]
    eb2 = [edge_b2[i].reshape(1, H) for i in range(L)]
    nw1a = [node_w1[i, :H].astype(bf16) for i in range(L)]
    nw1b = [node_w1[i, H:].astype(bf16) for i in range(L)]
    nb1 = [node_b1[i].reshape(1, H) for i in range(L)]
    nb2 = [node_b2[i].reshape(1, H) for i in range(L)]
    nw2 = [node_w2[i].astype(bf16) for i in range(L)]
    ew2 = [edge_w2[i].astype(bf16) for i in range(L)]
    w1s16 = [w.astype(bf16) for w in w1s]
    w1d16 = [w.astype(bf16) for w in w1d]

    hh, a, b = _node_first(h, emb_in_w.astype(bf16),
                           emb_in_b.reshape(1, H), w1s16[0], w1d16[0])
    h_out = None
    for i in range(L):
        s = _gather_add_sc(a, b, row, col)
        m = _edge_mlp(s, radial, edge_attr, w1rad[i], w1e[i],
                      eb1[i], ew2[i], eb2[i])
        agg2 = _scatter_sc(m, row, zeros_nh)
        if i < L - 1:
            hh, a, b = _node_mid(hh, agg2, nw1a[i], nw1b[i], nb1[i],
                                 nw2[i], nb2[i], w1s16[i + 1], w1d16[i + 1])
        else:
            h_out = _node_last(hh, agg2, nw1a[i], nw1b[i], nb1[i],
                               nw2[i], nb2[i], emb_out_w.astype(bf16),
                               emb_out_b.reshape(1, H))
    return (x, h_out)


# speedup vs baseline: 1.1115x; 1.1115x over previous
import functools

import jax
import jax.numpy as jnp
from jax import lax
from jax.experimental import pallas as pl
from jax.experimental.pallas import tpu as pltpu
from jax.experimental.pallas import tpu_sc as plsc

N = 10000
E = 320000
H = 128
NC = 2
NS = 16
NW = NC * NS
CH = 128

_mesh = plsc.VectorSubcoreMesh(core_axis_name="c", subcore_axis_name="s")

f32 = jnp.float32
bf16 = jnp.bfloat16
i32 = jnp.int32


def _silu(v):
    return v * (1.0 / (1.0 + jnp.exp(-v)))


def _bdot(x, w):
    return jnp.dot(x.astype(bf16), w, preferred_element_type=f32)


@functools.partial(
    pl.kernel,
    out_type=jax.ShapeDtypeStruct((E,), f32),
    mesh=_mesh,
    compiler_params=pltpu.CompilerParams(needs_layout_passes=False),
    scratch_types=[
        pltpu.VMEM((4, N), f32),
        pltpu.VMEM((CH,), i32),
        pltpu.VMEM((CH,), i32),
        pltpu.VMEM((CH,), f32),
    ],
)
def _radial_sc(xt_hbm, row_hbm, col_hbm, rad_hbm, xt_v, row_v, col_v, rad_v):
    cid = lax.axis_index("c")
    sid = lax.axis_index("s")
    wid = sid * NC + cid
    pltpu.sync_copy(xt_hbm, xt_v)
    nchunk = E // CH

    def body(j, carry):
        chunk = wid + j * NW

        @pl.when(chunk < nchunk)
        def _():
            base = chunk * CH
            pltpu.sync_copy(row_hbm.at[pl.ds(base, CH)], row_v)
            pltpu.sync_copy(col_hbm.at[pl.ds(base, CH)], col_v)

            def sub(k, c2):
                ridx = row_v[pl.ds(k * 16, 16)]
                cidx = col_v[pl.ds(k * 16, 16)]
                acc = jnp.zeros((16,), f32)
                for d in range(3):
                    didx = jnp.full((16,), d, i32)
                    xr = plsc.load_gather(xt_v, [didx, ridx])
                    xc = plsc.load_gather(xt_v, [didx, cidx])
                    dd = xr - xc
                    acc = acc + dd * dd
                rad_v[pl.ds(k * 16, 16)] = acc
                return c2

            lax.fori_loop(0, CH // 16, sub, 0)
            pltpu.sync_copy(rad_v, rad_hbm.at[pl.ds(base, CH)])

        return carry

    lax.fori_loop(0, (nchunk + NW - 1) // NW, body, 0)


NCHUNK = E // CH
PER_W = NCHUNK // NW
TAILC = NCHUNK - PER_W * NW
NITER = PER_W // 2


def _accum_rows(dst_v, src_v):
    def rowbody(r, c):
        for q in range(H // 16):
            sl = pl.ds(q * 16, 16)
            dst_v[r, sl] = dst_v[r, sl] + src_v[r, sl]
        return c

    lax.fori_loop(0, CH, rowbody, 0)


@functools.partial(
    pl.kernel,
    out_type=jax.ShapeDtypeStruct((E, H), f32),
    mesh=_mesh,
    scratch_types=[
        pltpu.VMEM((CH,), i32), pltpu.VMEM((CH,), i32),
        pltpu.VMEM((CH,), i32), pltpu.VMEM((CH,), i32),
        pltpu.VMEM((CH, H), f32), pltpu.VMEM((CH, H), f32),
        pltpu.VMEM((CH, H), f32), pltpu.VMEM((CH, H), f32),
        pltpu.SemaphoreType.DMA, pltpu.SemaphoreType.DMA,
        pltpu.SemaphoreType.DMA, pltpu.SemaphoreType.DMA,
        pltpu.SemaphoreType.DMA, pltpu.SemaphoreType.DMA,
    ],
)
def _gather_add_sc(a_hbm, b_hbm, row_hbm, col_hbm, s_hbm,
                   rv0, rv1, cv0, cv1, av0, av1, bv0, bv1,
                   si0, si1, sg0, sg1, sw0, sw1):
    cid = lax.axis_index("c")
    sid = lax.axis_index("s")
    wid = sid * NC + cid
    start = wid * PER_W
    rv = (rv0, rv1)
    cv = (cv0, cv1)
    av = (av0, av1)
    bv = (bv0, bv1)
    si = (si0, si1)
    sg = (sg0, sg1)
    sw = (sw0, sw1)

    def issue_idx(t, p):
        base = t * CH
        pltpu.async_copy(row_hbm.at[pl.ds(base, CH)], rv[p], si[p])
        pltpu.async_copy(col_hbm.at[pl.ds(base, CH)], cv[p], si[p])

    def wait_idx(p):
        pltpu.make_async_copy(row_hbm.at[pl.ds(0, CH)], rv[p], si[p]).wait()
        pltpu.make_async_copy(col_hbm.at[pl.ds(0, CH)], cv[p], si[p]).wait()

    def issue_gather(p):
        pltpu.async_copy(a_hbm.at[rv[p]], av[p], sg[p])
        pltpu.async_copy(b_hbm.at[cv[p]], bv[p], sg[p])

    def wait_gather(p):
        pltpu.make_async_copy(a_hbm.at[pl.ds(0, CH)], av[p], sg[p]).wait()
        pltpu.make_async_copy(b_hbm.at[pl.ds(0, CH)], bv[p], sg[p]).wait()

    def issue_write(t, p):
        pltpu.async_copy(av[p], s_hbm.at[pl.ds(t * CH, CH)], sw[p])

    def wait_write(p):
        pltpu.make_async_copy(a_hbm.at[pl.ds(0, CH)], av[p], sw[p]).wait()

    issue_idx(start, 0)
    wait_idx(0)
    issue_gather(0)

    def body(jj, carry):
        t0 = start + 2 * jj
        t1 = t0 + 1

        @pl.when(jj > 0)
        def _():
            wait_write(1)

        issue_idx(t1, 1)
        wait_idx(1)
        issue_gather(1)

        wait_gather(0)
        _accum_rows(av0, bv0)
        issue_write(t0, 0)

        @pl.when(jj + 1 < NITER)
        def _():
            wait_write(0)
            issue_idx(t0 + 2, 0)
            wait_idx(0)
            issue_gather(0)

        wait_gather(1)
        _accum_rows(av1, bv1)
        issue_write(t1, 1)
        return carry

    lax.fori_loop(0, NITER, body, 0)
    wait_write(0)
    wait_write(1)

    @pl.when(wid < TAILC)
    def _():
        t = NW * PER_W + wid
        issue_idx(t, 0)
        wait_idx(0)
        issue_gather(0)
        wait_gather(0)
        _accum_rows(av0, bv0)
        issue_write(t, 0)
        wait_write(0)


NP = 10240


SC_CHUNKS = (E // NC) // CH
SC_PER_T = SC_CHUNKS // NS
SC_TAIL = SC_CHUNKS - SC_PER_T * NS


@functools.partial(
    pl.kernel,
    out_type=jax.ShapeDtypeStruct((NC, NP, H), f32),
    mesh=_mesh,
    scratch_types=[
        pltpu.VMEM_SHARED((NP, H), f32),
        pltpu.VMEM((CH,), i32), pltpu.VMEM((CH,), i32),
        pltpu.VMEM((CH, H), f32), pltpu.VMEM((CH, H), f32),
        pltpu.SemaphoreType.DMA, pltpu.SemaphoreType.DMA,
        pltpu.SemaphoreType.DMA, pltpu.SemaphoreType.DMA,
    ],
)
def _scatter_sc(m_hbm, row_hbm, zero_hbm, out_hbm, acc_sh,
                iv0, iv1, mv0, mv1, sl0, sl1, ss0, ss1):
    cid = lax.axis_index("c")
    sid = lax.axis_index("s")
    rows_per_tile = NP // NS
    r0 = sid * rows_per_tile
    pltpu.sync_copy(zero_hbm.at[pl.ds(r0, rows_per_tile)],
                    acc_sh.at[pl.ds(r0, rows_per_tile)])
    plsc.subcore_barrier()

    iv = (iv0, iv1)
    mv = (mv0, mv1)
    sl = (sl0, sl1)
    ss = (ss0, ss1)
    start = cid * SC_CHUNKS + sid * SC_PER_T

    def issue_load(t, p):
        base = t * CH
        pltpu.async_copy(row_hbm.at[pl.ds(base, CH)], iv[p], sl[p])
        pltpu.async_copy(m_hbm.at[pl.ds(base, CH)], mv[p], sl[p])

    def wait_load(p):
        pltpu.make_async_copy(row_hbm.at[pl.ds(0, CH)], iv[p], sl[p]).wait()
        pltpu.make_async_copy(m_hbm.at[pl.ds(0, CH)], mv[p], sl[p]).wait()

    def issue_scat(p):
        pltpu.async_copy(mv[p], acc_sh.at[iv[p]], ss[p], add=True)

    def wait_scat(p):
        pltpu.make_async_copy(m_hbm.at[pl.ds(0, CH)], mv[p], ss[p]).wait()

    issue_load(start, 0)

    def body(jj, carry):
        t0 = start + 2 * jj
        t1 = t0 + 1

        @pl.when(jj > 0)
        def _():
            wait_scat(1)

        issue_load(t1, 1)
        wait_load(0)
        issue_scat(0)

        @pl.when(jj + 1 < SC_PER_T // 2)
        def _():
            wait_scat(0)
            issue_load(t0 + 2, 0)

        wait_load(1)
        issue_scat(1)
        return carry

    lax.fori_loop(0, SC_PER_T // 2, body, 0)
    wait_scat(0)
    wait_scat(1)

    @pl.when(sid < SC_TAIL)
    def _():
        t = cid * SC_CHUNKS + NS * SC_PER_T + sid
        issue_load(t, 0)
        wait_load(0)
        issue_scat(0)
        wait_scat(0)

    plsc.subcore_barrier()
    pltpu.sync_copy(acc_sh.at[pl.ds(r0, rows_per_tile)],
                    out_hbm.at[cid, pl.ds(r0, rows_per_tile)])


BN = 1000
BE = 2000


def _node_first_body(h_ref, wemb_ref, bemb_ref, w1s_ref, w1d_ref,
                     hh_ref, a_ref, b_ref):
    hh = _bdot(h_ref[...], wemb_ref[...])
    hh = hh + bemb_ref[...]
    hh_ref[...] = hh
    a_ref[...] = _bdot(hh, w1s_ref[...])
    b_ref[...] = _bdot(hh, w1d_ref[...])


def _node_mid_body(hh_ref, agg_ref, nw1a_ref, nw1b_ref, nb1_ref,
                   nw2_ref, nb2_ref, w1s_ref, w1d_ref,
                   hh_out_ref, a_ref, b_ref):
    hh = hh_ref[...]
    agg = agg_ref[0] + agg_ref[1]
    u = _silu(_bdot(hh, nw1a_ref[...])
              + _bdot(agg, nw1b_ref[...])
              + nb1_ref[...])
    out = _bdot(u, nw2_ref[...]) + nb2_ref[...]
    hh = hh + out
    hh_out_ref[...] = hh
    a_ref[...] = _bdot(hh, w1s_ref[...])
    b_ref[...] = _bdot(hh, w1d_ref[...])


def _node_last_body(hh_ref, agg_ref, nw1a_ref, nw1b_ref, nb1_ref,
                    nw2_ref, nb2_ref, wout_ref, bout_ref, hout_ref):
    hh = hh_ref[...]
    agg = agg_ref[0] + agg_ref[1]
    u = _silu(_bdot(hh, nw1a_ref[...])
              + _bdot(agg, nw1b_ref[...])
              + nb1_ref[...])
    out = _bdot(u, nw2_ref[...]) + nb2_ref[...]
    hh = hh + out
    hout_ref[...] = (_bdot(hh, wout_ref[...])
                     + bout_ref[...])


def _edge_body(s_ref, rad_ref, ea_ref, w1rad_ref, w1e_ref,
               b1_ref, w2_ref, b2_ref, m_ref):
    pre = (s_ref[...]
           + rad_ref[...] * w1rad_ref[...]
           + _bdot(ea_ref[...], w1e_ref[...])
           + b1_ref[...])
    t = _silu(pre)
    mm = _bdot(t, w2_ref[...]) + b2_ref[...]
    m_ref[...] = _silu(mm)


def _wspec(shape):
    return pl.BlockSpec(shape, lambda i: tuple(0 for _ in shape))


def _node_call(body, n_out, extra_in_specs):
    grid = N // BN
    rowspec = pl.BlockSpec((BN, H), lambda i: (i, 0))
    in_specs = [rowspec] + extra_in_specs
    out_specs = [rowspec] * n_out
    out_shape = [jax.ShapeDtypeStruct((N, H), f32)] * n_out
    if n_out == 1:
        out_specs = out_specs[0]
        out_shape = out_shape[0]
    return pl.pallas_call(body, grid=grid, in_specs=in_specs,
                          out_specs=out_specs, out_shape=out_shape)


_W = _wspec((H, H))
_BIAS = _wspec((1, H))
_AGGSPEC = pl.BlockSpec((NC, BN, H), lambda i: (0, i, 0))

_node_first = _node_call(_node_first_body, 3, [_W, _BIAS, _W, _W])
_node_mid = _node_call(_node_mid_body, 3,
                       [_AGGSPEC, _W, _W, _BIAS, _W, _BIAS, _W, _W])
_node_last = _node_call(_node_last_body, 1,
                        [_AGGSPEC, _W, _W, _BIAS, _W, _BIAS, _W, _BIAS])

_edge_mlp = pl.pallas_call(
    _edge_body,
    grid=E // BE,
    in_specs=[
        pl.BlockSpec((BE, H), lambda i: (i, 0)),
        pl.BlockSpec((BE, 1), lambda i: (i, 0)),
        pl.BlockSpec((BE, 16), lambda i: (i, 0)),
        _BIAS,
        _wspec((16, H)),
        _BIAS,
        _W,
        _BIAS,
    ],
    out_specs=pl.BlockSpec((BE, H), lambda i: (i, 0)),
    out_shape=jax.ShapeDtypeStruct((E, H), f32),
)


def kernel(h, x, edge_attr, emb_in_w, emb_in_b, edge_w1, edge_b1, edge_w2,
           edge_b2, node_w1, node_b1, node_w2, node_b2, emb_out_w, emb_out_b,
           edge_index):
    L = edge_w1.shape[0]
    row = edge_index[0]
    col = edge_index[1]

    xt = jnp.zeros((4, N), f32).at[:3].set(x.T)
    radial = _radial_sc(xt, row, col).reshape(E, 1)

    zeros_nh = jnp.zeros((NP, H), f32)

    w1s = [edge_w1[i, :H].astype(bf16) for i in range(L)]
    w1d = [edge_w1[i, H:2 * H].astype(bf16) for i in range(L)]
    w1rad = [edge_w1[i, 2 * H].reshape(1, H) for i in range(L)]
    w1e = [edge_w1[i, 2 * H + 1:].astype(bf16) for i in range(L)]
    eb1 = [edge_b1[i].reshape(1, H) for i in range(L)]
    eb2 = [edge_b2[i].reshape(1, H) for i in range(L)]
    nw1a = [node_w1[i, :H].astype(bf16) for i in range(L)]
    nw1b = [node_w1[i, H:].astype(bf16) for i in range(L)]
    nb1 = [node_b1[i].reshape(1, H) for i in range(L)]
    nb2 = [node_b2[i].reshape(1, H) for i in range(L)]
    nw2 = [node_w2[i].astype(bf16) for i in range(L)]
    ew2 = [edge_w2[i].astype(bf16) for i in range(L)]

    hh, a, b = _node_first(h, emb_in_w.astype(bf16), emb_in_b.reshape(1, H),
                           w1s[0], w1d[0])
    h_out = None
    for i in range(L):
        s = _gather_add_sc(a, b, row, col)
        m = _edge_mlp(s, radial, edge_attr, w1rad[i], w1e[i],
                      eb1[i], ew2[i], eb2[i])
        agg2 = _scatter_sc(m, row, zeros_nh)
        if i < L - 1:
            hh, a, b = _node_mid(hh, agg2, nw1a[i], nw1b[i], nb1[i],
                                 nw2[i], nb2[i], w1s[i + 1], w1d[i + 1])
        else:
            h_out = _node_last(hh, agg2, nw1a[i], nw1b[i], nb1[i],
                               nw2[i], nb2[i], emb_out_w.astype(bf16),
                               emb_out_b.reshape(1, H))
    return (x, h_out)


# speedup vs baseline: 1.1426x; 1.0280x over previous
import functools

import jax
import jax.numpy as jnp
from jax import lax
from jax.experimental import pallas as pl
from jax.experimental.pallas import tpu as pltpu
from jax.experimental.pallas import tpu_sc as plsc

N = 10000
E = 320000
E2 = E // 2
H = 128
NC = 2
NS = 16
NW = NC * NS
CH = 128

_mesh = plsc.VectorSubcoreMesh(core_axis_name="c", subcore_axis_name="s")

f32 = jnp.float32
bf16 = jnp.bfloat16
i32 = jnp.int32


def _silu(v):
    return v * (1.0 / (1.0 + jnp.exp(-v)))


def _bdot(x, w):
    return jnp.dot(x, w, preferred_element_type=f32)


@functools.partial(
    pl.kernel,
    out_type=jax.ShapeDtypeStruct((E,), f32),
    mesh=_mesh,
    compiler_params=pltpu.CompilerParams(needs_layout_passes=False),
    scratch_types=[
        pltpu.VMEM((4, N), f32),
        pltpu.VMEM((CH,), i32),
        pltpu.VMEM((CH,), i32),
        pltpu.VMEM((CH,), f32),
    ],
)
def _radial_sc(xt_hbm, row_hbm, col_hbm, rad_hbm, xt_v, row_v, col_v, rad_v):
    cid = lax.axis_index("c")
    sid = lax.axis_index("s")
    wid = sid * NC + cid
    pltpu.sync_copy(xt_hbm, xt_v)
    nchunk = E // CH

    def body(j, carry):
        chunk = wid + j * NW

        @pl.when(chunk < nchunk)
        def _():
            base = chunk * CH
            pltpu.sync_copy(row_hbm.at[pl.ds(base, CH)], row_v)
            pltpu.sync_copy(col_hbm.at[pl.ds(base, CH)], col_v)

            def sub(k, c2):
                ridx = row_v[pl.ds(k * 16, 16)]
                cidx = col_v[pl.ds(k * 16, 16)]
                acc = jnp.zeros((16,), f32)
                for d in range(3):
                    didx = jnp.full((16,), d, i32)
                    xr = plsc.load_gather(xt_v, [didx, ridx])
                    xc = plsc.load_gather(xt_v, [didx, cidx])
                    dd = xr - xc
                    acc = acc + dd * dd
                rad_v[pl.ds(k * 16, 16)] = acc
                return c2

            lax.fori_loop(0, CH // 16, sub, 0)
            pltpu.sync_copy(rad_v, rad_hbm.at[pl.ds(base, CH)])

        return carry

    lax.fori_loop(0, (nchunk + NW - 1) // NW, body, 0)


def _accum_rows(dst_v, src_v):
    def rowbody(r, c):
        for q in range(H // 16):
            sl = pl.ds(q * 16, 16)
            dst_v[r, sl] = dst_v[r, sl] + src_v[r, sl]
        return c

    lax.fori_loop(0, CH, rowbody, 0)


def _make_gather_add(esz):
    nchunk = esz // CH
    per_w = (nchunk // NW) & ~1
    niter = per_w // 2
    rem = nchunk - per_w * NW
    rem_rounds = -(-rem // NW)

    @functools.partial(
        pl.kernel,
        out_type=jax.ShapeDtypeStruct((esz, H), f32),
        mesh=_mesh,
        scratch_types=[
            pltpu.VMEM((CH,), i32), pltpu.VMEM((CH,), i32),
            pltpu.VMEM((CH,), i32), pltpu.VMEM((CH,), i32),
            pltpu.VMEM((CH, H), f32), pltpu.VMEM((CH, H), f32),
            pltpu.VMEM((CH, H), f32), pltpu.VMEM((CH, H), f32),
            pltpu.SemaphoreType.DMA, pltpu.SemaphoreType.DMA,
            pltpu.SemaphoreType.DMA, pltpu.SemaphoreType.DMA,
            pltpu.SemaphoreType.DMA, pltpu.SemaphoreType.DMA,
        ],
    )
    def gather_kernel(a_hbm, b_hbm, row_hbm, col_hbm, s_hbm,
                      rv0, rv1, cv0, cv1, av0, av1, bv0, bv1,
                      si0, si1, sg0, sg1, sw0, sw1):
        cid = lax.axis_index("c")
        sid = lax.axis_index("s")
        wid = sid * NC + cid
        start = wid * per_w
        rv = (rv0, rv1)
        cv = (cv0, cv1)
        av = (av0, av1)
        bv = (bv0, bv1)
        si = (si0, si1)
        sg = (sg0, sg1)
        sw = (sw0, sw1)

        def issue_idx(t, p):
            base = t * CH
            pltpu.async_copy(row_hbm.at[pl.ds(base, CH)], rv[p], si[p])
            pltpu.async_copy(col_hbm.at[pl.ds(base, CH)], cv[p], si[p])

        def wait_idx(p):
            pltpu.make_async_copy(row_hbm.at[pl.ds(0, CH)], rv[p],
                                  si[p]).wait()
            pltpu.make_async_copy(col_hbm.at[pl.ds(0, CH)], cv[p],
                                  si[p]).wait()

        def issue_gather(p):
            pltpu.async_copy(a_hbm.at[rv[p]], av[p], sg[p])
            pltpu.async_copy(b_hbm.at[cv[p]], bv[p], sg[p])

        def wait_gather(p):
            pltpu.make_async_copy(a_hbm.at[pl.ds(0, CH)], av[p],
                                  sg[p]).wait()
            pltpu.make_async_copy(b_hbm.at[pl.ds(0, CH)], bv[p],
                                  sg[p]).wait()

        def issue_write(t, p):
            pltpu.async_copy(av[p], s_hbm.at[pl.ds(t * CH, CH)], sw[p])

        def wait_write(p):
            pltpu.make_async_copy(a_hbm.at[pl.ds(0, CH)], av[p],
                                  sw[p]).wait()

        issue_idx(start, 0)
        wait_idx(0)
        issue_gather(0)

        def body(jj, carry):
            t0 = start + 2 * jj
            t1 = t0 + 1

            @pl.when(jj > 0)
            def _():
                wait_write(1)

            issue_idx(t1, 1)
            wait_idx(1)
            issue_gather(1)

            wait_gather(0)
            _accum_rows(av0, bv0)
            issue_write(t0, 0)

            @pl.when(jj + 1 < niter)
            def _():
                wait_write(0)
                issue_idx(t0 + 2, 0)
                wait_idx(0)
                issue_gather(0)

            wait_gather(1)
            _accum_rows(av1, bv1)
            issue_write(t1, 1)
            return carry

        lax.fori_loop(0, niter, body, 0)
        wait_write(0)
        wait_write(1)

        for k in range(rem_rounds):
            t = per_w * NW + k * NW + wid

            @pl.when(t < nchunk)
            def _():
                issue_idx(t, 0)
                wait_idx(0)
                issue_gather(0)
                wait_gather(0)
                _accum_rows(av0, bv0)
                issue_write(t, 0)
                wait_write(0)

    return gather_kernel


_gather_half = _make_gather_add(E2)


NP = 10112
SC_CHUNKS = E2 // CH
SC_PER_T = (SC_CHUNKS // NS) & ~1
SC_TAIL = SC_CHUNKS - SC_PER_T * NS


@functools.partial(
    pl.kernel,
    out_type=jax.ShapeDtypeStruct((NC, NP, H), f32),
    mesh=_mesh,
    scratch_types=[
        pltpu.VMEM_SHARED((NP, H), f32),
        pltpu.VMEM((CH,), i32), pltpu.VMEM((CH,), i32),
        pltpu.VMEM((CH, H), f32), pltpu.VMEM((CH, H), f32),
        pltpu.SemaphoreType.DMA, pltpu.SemaphoreType.DMA,
        pltpu.SemaphoreType.DMA, pltpu.SemaphoreType.DMA,
    ],
)
def _scatter2_sc(m0_hbm, m1_hbm, row0_hbm, row1_hbm, zero_hbm, out_hbm,
                 acc_sh, iv0, iv1, mv0, mv1, sl0, sl1, ss0, ss1):
    cid = lax.axis_index("c")
    sid = lax.axis_index("s")
    rows_per_tile = NP // NS
    r0 = sid * rows_per_tile
    pltpu.sync_copy(zero_hbm.at[pl.ds(r0, rows_per_tile)],
                    acc_sh.at[pl.ds(r0, rows_per_tile)])
    plsc.subcore_barrier()

    iv = (iv0, iv1)
    mv = (mv0, mv1)
    sl = (sl0, sl1)
    ss = (ss0, ss1)

    def run_half(m_hbm, row_hbm):
        start = sid * SC_PER_T

        def issue_load(t, p):
            base = t * CH
            pltpu.async_copy(row_hbm.at[pl.ds(base, CH)], iv[p], sl[p])
            pltpu.async_copy(m_hbm.at[pl.ds(base, CH)], mv[p], sl[p])

        def wait_load(p):
            pltpu.make_async_copy(row_hbm.at[pl.ds(0, CH)], iv[p],
                                  sl[p]).wait()
            pltpu.make_async_copy(m_hbm.at[pl.ds(0, CH)], mv[p],
                                  sl[p]).wait()

        def issue_scat(p):
            pltpu.async_copy(mv[p], acc_sh.at[iv[p]], ss[p], add=True)

        def wait_scat(p):
            pltpu.make_async_copy(m_hbm.at[pl.ds(0, CH)], mv[p],
                                  ss[p]).wait()

        issue_load(start, 0)

        def body(jj, carry):
            t0 = start + 2 * jj
            t1 = t0 + 1

            @pl.when(jj > 0)
            def _():
                wait_scat(1)

            issue_load(t1, 1)
            wait_load(0)
            issue_scat(0)

            @pl.when(jj + 1 < SC_PER_T // 2)
            def _():
                wait_scat(0)
                issue_load(t0 + 2, 0)

            wait_load(1)
            issue_scat(1)
            return carry

        lax.fori_loop(0, SC_PER_T // 2, body, 0)
        wait_scat(0)
        wait_scat(1)

        @pl.when(sid < SC_TAIL)
        def _():
            t = NS * SC_PER_T + sid
            issue_load(t, 0)
            wait_load(0)
            issue_scat(0)
            wait_scat(0)

    @pl.when(cid == 0)
    def _():
        run_half(m0_hbm, row0_hbm)

    @pl.when(cid == 1)
    def _():
        run_half(m1_hbm, row1_hbm)

    plsc.subcore_barrier()
    pltpu.sync_copy(acc_sh.at[pl.ds(r0, rows_per_tile)],
                    out_hbm.at[cid, pl.ds(r0, rows_per_tile)])


BN = 1000
BE = 2000


def _node_first_body(h_ref, wemb_ref, bemb_ref, w1s_ref, w1d_ref,
                     hh_ref, a_ref, b_ref):
    hh = _bdot(h_ref[...], wemb_ref[...]) + bemb_ref[...]
    hh_ref[...] = hh
    a_ref[...] = _bdot(hh, w1s_ref[...])
    b_ref[...] = _bdot(hh, w1d_ref[...])


def _node_mid_body(hh_ref, agg_ref, nw1a_ref, nw1b_ref, nb1_ref,
                   nw2_ref, nb2_ref, w1s_ref, w1d_ref,
                   hh_out_ref, a_ref, b_ref):
    hh = hh_ref[...]
    agg = agg_ref[0] + agg_ref[1]
    u = _silu(_bdot(hh, nw1a_ref[...]) + _bdot(agg, nw1b_ref[...])
              + nb1_ref[...])
    out = _bdot(u, nw2_ref[...]) + nb2_ref[...]
    hh = hh + out
    hh_out_ref[...] = hh
    a_ref[...] = _bdot(hh, w1s_ref[...])
    b_ref[...] = _bdot(hh, w1d_ref[...])


def _node_last_body(hh_ref, agg_ref, nw1a_ref, nw1b_ref, nb1_ref,
                    nw2_ref, nb2_ref, wout_ref, bout_ref, hout_ref):
    hh = hh_ref[...]
    agg = agg_ref[0] + agg_ref[1]
    u = _silu(_bdot(hh, nw1a_ref[...]) + _bdot(agg, nw1b_ref[...])
              + nb1_ref[...])
    out = _bdot(u, nw2_ref[...]) + nb2_ref[...]
    hh = hh + out
    hout_ref[...] = _bdot(hh, wout_ref[...]) + bout_ref[...]


def _edge_body(s_ref, rad_ref, ea_ref, w1rad_ref, w1e_ref,
               b1_ref, w2_ref, b2_ref, m_ref):
    pre = (s_ref[...]
           + rad_ref[...] * w1rad_ref[...]
           + _bdot(ea_ref[...], w1e_ref[...])
           + b1_ref[...])
    t = _silu(pre)
    mm = _bdot(t, w2_ref[...]) + b2_ref[...]
    m_ref[...] = _silu(mm)


def _wspec(shape):
    return pl.BlockSpec(shape, lambda i: tuple(0 for _ in shape))


def _node_call(body, out_dtypes, extra_in_specs):
    grid = N // BN
    rowspec = pl.BlockSpec((BN, H), lambda i: (i, 0))
    in_specs = [rowspec] + extra_in_specs
    out_specs = [rowspec] * len(out_dtypes)
    out_shape = [jax.ShapeDtypeStruct((N, H), dt) for dt in out_dtypes]
    if len(out_dtypes) == 1:
        out_specs = out_specs[0]
        out_shape = out_shape[0]
    return pl.pallas_call(body, grid=grid, in_specs=in_specs,
                          out_specs=out_specs, out_shape=out_shape)


_W = _wspec((H, H))
_BIAS = _wspec((1, H))
_AGGSPEC = pl.BlockSpec((NC, BN, H), lambda i: (0, i, 0))

_node_first = _node_call(_node_first_body, [f32, f32, f32],
                         [_W, _BIAS, _W, _W])
_node_mid = _node_call(_node_mid_body, [f32, f32, f32],
                       [_AGGSPEC, _W, _W, _BIAS, _W, _BIAS, _W, _W])
_node_last = _node_call(_node_last_body, [f32],
                        [_AGGSPEC, _W, _W, _BIAS, _W, _BIAS, _W, _BIAS])

_edge_mlp_half = pl.pallas_call(
    _edge_body,
    grid=E2 // BE,
    in_specs=[
        pl.BlockSpec((BE, H), lambda i: (i, 0)),
        pl.BlockSpec((BE, 1), lambda i: (i, 0)),
        pl.BlockSpec((BE, 16), lambda i: (i, 0)),
        _BIAS,
        _wspec((16, H)),
        _BIAS,
        _W,
        _BIAS,
    ],
    out_specs=pl.BlockSpec((BE, H), lambda i: (i, 0)),
    out_shape=jax.ShapeDtypeStruct((E2, H), f32),
)


def kernel(h, x, edge_attr, emb_in_w, emb_in_b, edge_w1, edge_b1, edge_w2,
           edge_b2, node_w1, node_b1, node_w2, node_b2, emb_out_w, emb_out_b,
           edge_index):
    L = edge_w1.shape[0]
    row = edge_index[0]
    col = edge_index[1]

    xt = jnp.zeros((4, N), f32).at[:3].set(x.T)
    radial = _radial_sc(xt, row, col).reshape(E, 1)

    zeros_nh = jnp.zeros((NP, H), f32)

    rows = (row[:E2], row[E2:])
    cols = (col[:E2], col[E2:])
    rads = (radial[:E2], radial[E2:])
    eas = (edge_attr[:E2], edge_attr[E2:])

    w1s = [edge_w1[i, :H] for i in range(L)]
    w1d = [edge_w1[i, H:2 * H] for i in range(L)]
    w1rad = [edge_w1[i, 2 * H].reshape(1, H) for i in range(L)]
    w1e = [edge_w1[i, 2 * H + 1:] for i in range(L)]
    eb1 = [edge_b1[i].reshape(1, H) for i in range(L)]
    eb2 = [edge_b2[i].reshape(1, H) for i in range(L)]
    nw1a = [node_w1[i, :H] for i in range(L)]
    nw1b = [node_w1[i, H:] for i in range(L)]
    nb1 = [node_b1[i].reshape(1, H) for i in range(L)]
    nb2 = [node_b2[i].reshape(1, H) for i in range(L)]
    nw2 = [node_w2[i] for i in range(L)]
    ew2 = [edge_w2[i] for i in range(L)]

    hh, a, b = _node_first(h, emb_in_w, emb_in_b.reshape(1, H),
                           w1s[0], w1d[0])
    h_out = None
    for i in range(L):
        m = [None, None]
        for k in range(2):
            s = _gather_half(a, b, rows[k], cols[k])
            m[k] = _edge_mlp_half(s, rads[k], eas[k], w1rad[i], w1e[i],
                                  eb1[i], ew2[i], eb2[i])
        agg2 = _scatter2_sc(m[0], m[1], rows[0], rows[1], zeros_nh)
        if i < L - 1:
            hh, a, b = _node_mid(hh, agg2, nw1a[i], nw1b[i], nb1[i],
                                 nw2[i], nb2[i], w1s[i + 1], w1d[i + 1])
        else:
            h_out = _node_last(hh, agg2, nw1a[i], nw1b[i], nb1[i],
                               nw2[i], nb2[i], emb_out_w,
                               emb_out_b.reshape(1, H))
    return (x, h_out)


# speedup vs baseline: 1.1507x; 1.0071x over previous
import functools

import jax
import jax.numpy as jnp
from jax import lax
from jax.experimental import pallas as pl
from jax.experimental.pallas import tpu as pltpu
from jax.experimental.pallas import tpu_sc as plsc

N = 10000
E = 320000
E2 = E // 2
H = 128
NC = 2
NS = 16
NW = NC * NS
CH = 128

_mesh = plsc.VectorSubcoreMesh(core_axis_name="c", subcore_axis_name="s")

f32 = jnp.float32
bf16 = jnp.bfloat16
i32 = jnp.int32


def _silu(v):
    return v * (1.0 / (1.0 + jnp.exp(-v)))


def _bdot(x, w):
    return jnp.dot(x, w, preferred_element_type=f32)


@functools.partial(
    pl.kernel,
    out_type=jax.ShapeDtypeStruct((E,), f32),
    mesh=_mesh,
    compiler_params=pltpu.CompilerParams(needs_layout_passes=False),
    scratch_types=[
        pltpu.VMEM((4, N), f32),
        pltpu.VMEM((CH,), i32),
        pltpu.VMEM((CH,), i32),
        pltpu.VMEM((CH,), f32),
    ],
)
def _radial_sc(xt_hbm, row_hbm, col_hbm, rad_hbm, xt_v, row_v, col_v, rad_v):
    cid = lax.axis_index("c")
    sid = lax.axis_index("s")
    wid = sid * NC + cid
    pltpu.sync_copy(xt_hbm, xt_v)
    nchunk = E // CH

    def body(j, carry):
        chunk = wid + j * NW

        @pl.when(chunk < nchunk)
        def _():
            base = chunk * CH
            pltpu.sync_copy(row_hbm.at[pl.ds(base, CH)], row_v)
            pltpu.sync_copy(col_hbm.at[pl.ds(base, CH)], col_v)

            def sub(k, c2):
                ridx = row_v[pl.ds(k * 16, 16)]
                cidx = col_v[pl.ds(k * 16, 16)]
                acc = jnp.zeros((16,), f32)
                for d in range(3):
                    didx = jnp.full((16,), d, i32)
                    xr = plsc.load_gather(xt_v, [didx, ridx])
                    xc = plsc.load_gather(xt_v, [didx, cidx])
                    dd = xr - xc
                    acc = acc + dd * dd
                rad_v[pl.ds(k * 16, 16)] = acc
                return c2

            lax.fori_loop(0, CH // 16, sub, 0)
            pltpu.sync_copy(rad_v, rad_hbm.at[pl.ds(base, CH)])

        return carry

    lax.fori_loop(0, (nchunk + NW - 1) // NW, body, 0)


def _accum_rows(dst_v, src_v):
    def rowbody(r, c):
        for q in range(H // 16):
            sl = pl.ds(q * 16, 16)
            dst_v[r, sl] = dst_v[r, sl] + src_v[r, sl]
        return c

    lax.fori_loop(0, CH, rowbody, 0)


def _make_gather_add(esz):
    nchunk = esz // CH
    per_w = (nchunk // NW) & ~1
    niter = per_w // 2
    rem = nchunk - per_w * NW
    rem_rounds = -(-rem // NW)

    @functools.partial(
        pl.kernel,
        out_type=jax.ShapeDtypeStruct((esz, H), f32),
        mesh=_mesh,
        scratch_types=[
            pltpu.VMEM((CH,), i32), pltpu.VMEM((CH,), i32),
            pltpu.VMEM((CH,), i32), pltpu.VMEM((CH,), i32),
            pltpu.VMEM((CH, H), f32), pltpu.VMEM((CH, H), f32),
            pltpu.VMEM((CH, H), f32), pltpu.VMEM((CH, H), f32),
            pltpu.SemaphoreType.DMA, pltpu.SemaphoreType.DMA,
            pltpu.SemaphoreType.DMA, pltpu.SemaphoreType.DMA,
            pltpu.SemaphoreType.DMA, pltpu.SemaphoreType.DMA,
        ],
    )
    def gather_kernel(a_hbm, b_hbm, row_hbm, col_hbm, s_hbm,
                      rv0, rv1, cv0, cv1, av0, av1, bv0, bv1,
                      si0, si1, sg0, sg1, sw0, sw1):
        cid = lax.axis_index("c")
        sid = lax.axis_index("s")
        wid = sid * NC + cid
        start = wid * per_w
        rv = (rv0, rv1)
        cv = (cv0, cv1)
        av = (av0, av1)
        bv = (bv0, bv1)
        si = (si0, si1)
        sg = (sg0, sg1)
        sw = (sw0, sw1)

        def issue_idx(t, p):
            base = t * CH
            pltpu.async_copy(row_hbm.at[pl.ds(base, CH)], rv[p], si[p])
            pltpu.async_copy(col_hbm.at[pl.ds(base, CH)], cv[p], si[p])

        def wait_idx(p):
            pltpu.make_async_copy(row_hbm.at[pl.ds(0, CH)], rv[p],
                                  si[p]).wait()
            pltpu.make_async_copy(col_hbm.at[pl.ds(0, CH)], cv[p],
                                  si[p]).wait()

        def issue_gather(p):
            pltpu.async_copy(a_hbm.at[rv[p]], av[p], sg[p])
            pltpu.async_copy(b_hbm.at[cv[p]], bv[p], sg[p])

        def wait_gather(p):
            pltpu.make_async_copy(a_hbm.at[pl.ds(0, CH)], av[p],
                                  sg[p]).wait()
            pltpu.make_async_copy(b_hbm.at[pl.ds(0, CH)], bv[p],
                                  sg[p]).wait()

        def issue_write(t, p):
            pltpu.async_copy(av[p], s_hbm.at[pl.ds(t * CH, CH)], sw[p])

        def wait_write(p):
            pltpu.make_async_copy(a_hbm.at[pl.ds(0, CH)], av[p],
                                  sw[p]).wait()

        issue_idx(start, 0)
        wait_idx(0)
        issue_gather(0)

        def body(jj, carry):
            t0 = start + 2 * jj
            t1 = t0 + 1

            @pl.when(jj > 0)
            def _():
                wait_write(1)

            issue_idx(t1, 1)
            wait_idx(1)
            issue_gather(1)

            wait_gather(0)
            _accum_rows(av0, bv0)
            issue_write(t0, 0)

            @pl.when(jj + 1 < niter)
            def _():
                wait_write(0)
                issue_idx(t0 + 2, 0)
                wait_idx(0)
                issue_gather(0)

            wait_gather(1)
            _accum_rows(av1, bv1)
            issue_write(t1, 1)
            return carry

        lax.fori_loop(0, niter, body, 0)
        wait_write(0)
        wait_write(1)

        for k in range(rem_rounds):
            t = per_w * NW + k * NW + wid

            @pl.when(t < nchunk)
            def _():
                issue_idx(t, 0)
                wait_idx(0)
                issue_gather(0)
                wait_gather(0)
                _accum_rows(av0, bv0)
                issue_write(t, 0)
                wait_write(0)

    return gather_kernel


_gather_half = _make_gather_add(E2)


NP = 10112
SC_CHUNKS = E2 // (NC * CH)
SC_PER_T = (SC_CHUNKS // NS) & ~1
SC_REM = SC_CHUNKS - SC_PER_T * NS
SC_ROUNDS = -(-SC_REM // NS)


@functools.partial(
    pl.kernel,
    out_type=jax.ShapeDtypeStruct((NC, NP, H), f32),
    mesh=_mesh,
    scratch_types=[
        pltpu.VMEM_SHARED((NP, H), f32),
        pltpu.VMEM((CH,), i32), pltpu.VMEM((CH,), i32),
        pltpu.VMEM((CH, H), f32), pltpu.VMEM((CH, H), f32),
        pltpu.SemaphoreType.DMA, pltpu.SemaphoreType.DMA,
        pltpu.SemaphoreType.DMA, pltpu.SemaphoreType.DMA,
    ],
)
def _scatter_half_sc(m_hbm, row_hbm, zero_hbm, out_hbm,
                     acc_sh, iv0, iv1, mv0, mv1, sl0, sl1, ss0, ss1):
    cid = lax.axis_index("c")
    sid = lax.axis_index("s")
    rows_per_tile = NP // NS
    r0 = sid * rows_per_tile
    pltpu.sync_copy(zero_hbm.at[pl.ds(r0, rows_per_tile)],
                    acc_sh.at[pl.ds(r0, rows_per_tile)])
    plsc.subcore_barrier()

    iv = (iv0, iv1)
    mv = (mv0, mv1)
    sl = (sl0, sl1)
    ss = (ss0, ss1)
    core0 = cid * SC_CHUNKS
    start = core0 + sid * SC_PER_T

    def issue_load(t, p):
        base = t * CH
        pltpu.async_copy(row_hbm.at[pl.ds(base, CH)], iv[p], sl[p])
        pltpu.async_copy(m_hbm.at[pl.ds(base, CH)], mv[p], sl[p])

    def wait_load(p):
        pltpu.make_async_copy(row_hbm.at[pl.ds(0, CH)], iv[p],
                              sl[p]).wait()
        pltpu.make_async_copy(m_hbm.at[pl.ds(0, CH)], mv[p],
                              sl[p]).wait()

    def issue_scat(p):
        pltpu.async_copy(mv[p], acc_sh.at[iv[p]], ss[p], add=True)

    def wait_scat(p):
        pltpu.make_async_copy(m_hbm.at[pl.ds(0, CH)], mv[p],
                              ss[p]).wait()

    issue_load(start, 0)

    def body(jj, carry):
        t0 = start + 2 * jj
        t1 = t0 + 1

        @pl.when(jj > 0)
        def _():
            wait_scat(1)

        issue_load(t1, 1)
        wait_load(0)
        issue_scat(0)

        @pl.when(jj + 1 < SC_PER_T // 2)
        def _():
            wait_scat(0)
            issue_load(t0 + 2, 0)

        wait_load(1)
        issue_scat(1)
        return carry

    lax.fori_loop(0, SC_PER_T // 2, body, 0)
    wait_scat(0)
    wait_scat(1)

    for k in range(SC_ROUNDS):
        t = core0 + NS * SC_PER_T + k * NS + sid

        @pl.when(t < core0 + SC_CHUNKS)
        def _():
            issue_load(t, 0)
            wait_load(0)
            issue_scat(0)
            wait_scat(0)

    plsc.subcore_barrier()
    pltpu.sync_copy(acc_sh.at[pl.ds(r0, rows_per_tile)],
                    out_hbm.at[cid, pl.ds(r0, rows_per_tile)])


BN = 1000
BE = 2000


def _node_first_body(h_ref, wemb_ref, bemb_ref, w1s_ref, w1d_ref,
                     hh_ref, a_ref, b_ref):
    hh = _bdot(h_ref[...], wemb_ref[...]) + bemb_ref[...]
    hh_ref[...] = hh
    a_ref[...] = _bdot(hh, w1s_ref[...])
    b_ref[...] = _bdot(hh, w1d_ref[...])


def _node_mid_body(hh_ref, agga_ref, aggb_ref, nw1a_ref, nw1b_ref, nb1_ref,
                   nw2_ref, nb2_ref, w1s_ref, w1d_ref,
                   hh_out_ref, a_ref, b_ref):
    hh = hh_ref[...]
    agg = (agga_ref[0] + agga_ref[1]) + (aggb_ref[0] + aggb_ref[1])
    u = _silu(_bdot(hh, nw1a_ref[...]) + _bdot(agg, nw1b_ref[...])
              + nb1_ref[...])
    out = _bdot(u, nw2_ref[...]) + nb2_ref[...]
    hh = hh + out
    hh_out_ref[...] = hh
    a_ref[...] = _bdot(hh, w1s_ref[...])
    b_ref[...] = _bdot(hh, w1d_ref[...])


def _node_last_body(hh_ref, agga_ref, aggb_ref, nw1a_ref, nw1b_ref, nb1_ref,
                    nw2_ref, nb2_ref, wout_ref, bout_ref, hout_ref):
    hh = hh_ref[...]
    agg = (agga_ref[0] + agga_ref[1]) + (aggb_ref[0] + aggb_ref[1])
    u = _silu(_bdot(hh, nw1a_ref[...]) + _bdot(agg, nw1b_ref[...])
              + nb1_ref[...])
    out = _bdot(u, nw2_ref[...]) + nb2_ref[...]
    hh = hh + out
    hout_ref[...] = _bdot(hh, wout_ref[...]) + bout_ref[...]


def _edge_body(s_ref, rad_ref, ea_ref, w1rad_ref, w1e_ref,
               b1_ref, w2_ref, b2_ref, m_ref):
    pre = (s_ref[...]
           + rad_ref[...] * w1rad_ref[...]
           + _bdot(ea_ref[...], w1e_ref[...])
           + b1_ref[...])
    t = _silu(pre)
    mm = _bdot(t, w2_ref[...]) + b2_ref[...]
    m_ref[...] = _silu(mm)


def _wspec(shape):
    return pl.BlockSpec(shape, lambda i: tuple(0 for _ in shape))


def _node_call(body, out_dtypes, extra_in_specs):
    grid = N // BN
    rowspec = pl.BlockSpec((BN, H), lambda i: (i, 0))
    in_specs = [rowspec] + extra_in_specs
    out_specs = [rowspec] * len(out_dtypes)
    out_shape = [jax.ShapeDtypeStruct((N, H), dt) for dt in out_dtypes]
    if len(out_dtypes) == 1:
        out_specs = out_specs[0]
        out_shape = out_shape[0]
    return pl.pallas_call(body, grid=grid, in_specs=in_specs,
                          out_specs=out_specs, out_shape=out_shape)


_W = _wspec((H, H))
_BIAS = _wspec((1, H))
_AGGSPEC = pl.BlockSpec((NC, BN, H), lambda i: (0, i, 0))

_node_first = _node_call(_node_first_body, [f32, f32, f32],
                         [_W, _BIAS, _W, _W])
_node_mid = _node_call(_node_mid_body, [f32, f32, f32],
                       [_AGGSPEC, _AGGSPEC, _W, _W, _BIAS, _W, _BIAS, _W, _W])
_node_last = _node_call(_node_last_body, [f32],
                        [_AGGSPEC, _AGGSPEC, _W, _W, _BIAS, _W, _BIAS, _W,
                         _BIAS])

_edge_mlp_half = pl.pallas_call(
    _edge_body,
    grid=E2 // BE,
    in_specs=[
        pl.BlockSpec((BE, H), lambda i: (i, 0)),
        pl.BlockSpec((BE, 1), lambda i: (i, 0)),
        pl.BlockSpec((BE, 16), lambda i: (i, 0)),
        _BIAS,
        _wspec((16, H)),
        _BIAS,
        _W,
        _BIAS,
    ],
    out_specs=pl.BlockSpec((BE, H), lambda i: (i, 0)),
    out_shape=jax.ShapeDtypeStruct((E2, H), f32),
)


def kernel(h, x, edge_attr, emb_in_w, emb_in_b, edge_w1, edge_b1, edge_w2,
           edge_b2, node_w1, node_b1, node_w2, node_b2, emb_out_w, emb_out_b,
           edge_index):
    L = edge_w1.shape[0]
    row = edge_index[0]
    col = edge_index[1]

    xt = jnp.zeros((4, N), f32).at[:3].set(x.T)
    radial = _radial_sc(xt, row, col).reshape(E, 1)

    zeros_nh = jnp.zeros((NP, H), f32)

    rows = (row[:E2], row[E2:])
    cols = (col[:E2], col[E2:])
    rads = (radial[:E2], radial[E2:])
    eas = (edge_attr[:E2], edge_attr[E2:])

    w1s = [edge_w1[i, :H] for i in range(L)]
    w1d = [edge_w1[i, H:2 * H] for i in range(L)]
    w1rad = [edge_w1[i, 2 * H].reshape(1, H) for i in range(L)]
    w1e = [edge_w1[i, 2 * H + 1:] for i in range(L)]
    eb1 = [edge_b1[i].reshape(1, H) for i in range(L)]
    eb2 = [edge_b2[i].reshape(1, H) for i in range(L)]
    nw1a = [node_w1[i, :H] for i in range(L)]
    nw1b = [node_w1[i, H:] for i in range(L)]
    nb1 = [node_b1[i].reshape(1, H) for i in range(L)]
    nb2 = [node_b2[i].reshape(1, H) for i in range(L)]
    nw2 = [node_w2[i] for i in range(L)]
    ew2 = [edge_w2[i] for i in range(L)]

    hh, a, b = _node_first(h, emb_in_w, emb_in_b.reshape(1, H),
                           w1s[0], w1d[0])
    h_out = None
    for i in range(L):
        m = [None, None]
        for k in range(2):
            s = _gather_half(a, b, rows[k], cols[k])
            m[k] = _edge_mlp_half(s, rads[k], eas[k], w1rad[i], w1e[i],
                                  eb1[i], ew2[i], eb2[i])
        agga = _scatter_half_sc(m[0], rows[0], zeros_nh)
        aggb = _scatter_half_sc(m[1], rows[1], zeros_nh)
        if i < L - 1:
            hh, a, b = _node_mid(hh, agga, aggb, nw1a[i], nw1b[i], nb1[i],
                                 nw2[i], nb2[i], w1s[i + 1], w1d[i + 1])
        else:
            h_out = _node_last(hh, agga, aggb, nw1a[i], nw1b[i], nb1[i],
                               nw2[i], nb2[i], emb_out_w,
                               emb_out_b.reshape(1, H))
    return (x, h_out)


# speedup vs baseline: 1.1966x; 1.0399x over previous
import functools

import jax
import jax.numpy as jnp
from jax import lax
from jax.experimental import pallas as pl
from jax.experimental.pallas import tpu as pltpu
from jax.experimental.pallas import tpu_sc as plsc

N = 10000
E = 320000
E2 = E // 2
H = 128
NC = 2
NS = 16
NW = NC * NS
CH = 128

_mesh = plsc.VectorSubcoreMesh(core_axis_name="c", subcore_axis_name="s")

f32 = jnp.float32
bf16 = jnp.bfloat16
i32 = jnp.int32


def _silu(v):
    return v * (1.0 / (1.0 + jnp.exp(-v)))


def _bdot(x, w):
    return jnp.dot(x, w, preferred_element_type=f32)


@functools.partial(
    pl.kernel,
    out_type=jax.ShapeDtypeStruct((E,), f32),
    mesh=_mesh,
    compiler_params=pltpu.CompilerParams(needs_layout_passes=False),
    scratch_types=[
        pltpu.VMEM((4, N), f32),
        pltpu.VMEM((CH,), i32),
        pltpu.VMEM((CH,), i32),
        pltpu.VMEM((CH,), f32),
    ],
)
def _radial_sc(xt_hbm, row_hbm, col_hbm, rad_hbm, xt_v, row_v, col_v, rad_v):
    cid = lax.axis_index("c")
    sid = lax.axis_index("s")
    wid = sid * NC + cid
    pltpu.sync_copy(xt_hbm, xt_v)
    nchunk = E // CH

    def body(j, carry):
        chunk = wid + j * NW

        @pl.when(chunk < nchunk)
        def _():
            base = chunk * CH
            pltpu.sync_copy(row_hbm.at[pl.ds(base, CH)], row_v)
            pltpu.sync_copy(col_hbm.at[pl.ds(base, CH)], col_v)

            def sub(k, c2):
                ridx = row_v[pl.ds(k * 16, 16)]
                cidx = col_v[pl.ds(k * 16, 16)]
                acc = jnp.zeros((16,), f32)
                for d in range(3):
                    didx = jnp.full((16,), d, i32)
                    xr = plsc.load_gather(xt_v, [didx, ridx])
                    xc = plsc.load_gather(xt_v, [didx, cidx])
                    dd = xr - xc
                    acc = acc + dd * dd
                rad_v[pl.ds(k * 16, 16)] = acc
                return c2

            lax.fori_loop(0, CH // 16, sub, 0)
            pltpu.sync_copy(rad_v, rad_hbm.at[pl.ds(base, CH)])

        return carry

    lax.fori_loop(0, (nchunk + NW - 1) // NW, body, 0)


def _accum_rows(dst_v, src_v):
    def rowbody(r, c):
        for q in range(H // 16):
            sl = pl.ds(q * 16, 16)
            dst_v[r, sl] = dst_v[r, sl] + src_v[r, sl]
        return c

    lax.fori_loop(0, CH, rowbody, 0)


def _make_gather_add(esz):
    nchunk = esz // CH
    per_w = (nchunk // NW) & ~1
    niter = per_w // 2
    rem = nchunk - per_w * NW
    rem_rounds = -(-rem // NW)

    @functools.partial(
        pl.kernel,
        out_type=jax.ShapeDtypeStruct((esz, H), f32),
        mesh=_mesh,
        scratch_types=[
            pltpu.VMEM((CH,), i32), pltpu.VMEM((CH,), i32),
            pltpu.VMEM((CH,), i32), pltpu.VMEM((CH,), i32),
            pltpu.VMEM((CH, H), f32), pltpu.VMEM((CH, H), f32),
            pltpu.VMEM((CH, H), f32), pltpu.VMEM((CH, H), f32),
            pltpu.SemaphoreType.DMA, pltpu.SemaphoreType.DMA,
            pltpu.SemaphoreType.DMA, pltpu.SemaphoreType.DMA,
            pltpu.SemaphoreType.DMA, pltpu.SemaphoreType.DMA,
        ],
    )
    def gather_kernel(a_hbm, b_hbm, row_hbm, col_hbm, s_hbm,
                      rv0, rv1, cv0, cv1, av0, av1, bv0, bv1,
                      si0, si1, sg0, sg1, sw0, sw1):
        cid = lax.axis_index("c")
        sid = lax.axis_index("s")
        wid = sid * NC + cid
        start = wid * per_w
        rv = (rv0, rv1)
        cv = (cv0, cv1)
        av = (av0, av1)
        bv = (bv0, bv1)
        si = (si0, si1)
        sg = (sg0, sg1)
        sw = (sw0, sw1)

        def issue_idx(t, p):
            base = t * CH
            pltpu.async_copy(row_hbm.at[pl.ds(base, CH)], rv[p], si[p])
            pltpu.async_copy(col_hbm.at[pl.ds(base, CH)], cv[p], si[p])

        def wait_idx(p):
            pltpu.make_async_copy(row_hbm.at[pl.ds(0, CH)], rv[p],
                                  si[p]).wait()
            pltpu.make_async_copy(col_hbm.at[pl.ds(0, CH)], cv[p],
                                  si[p]).wait()

        def issue_gather(p):
            pltpu.async_copy(a_hbm.at[rv[p]], av[p], sg[p])
            pltpu.async_copy(b_hbm.at[cv[p]], bv[p], sg[p])

        def wait_gather(p):
            pltpu.make_async_copy(a_hbm.at[pl.ds(0, CH)], av[p],
                                  sg[p]).wait()
            pltpu.make_async_copy(b_hbm.at[pl.ds(0, CH)], bv[p],
                                  sg[p]).wait()

        def issue_write(t, p):
            pltpu.async_copy(av[p], s_hbm.at[pl.ds(t * CH, CH)], sw[p])

        def wait_write(p):
            pltpu.make_async_copy(a_hbm.at[pl.ds(0, CH)], av[p],
                                  sw[p]).wait()

        issue_idx(start, 0)
        wait_idx(0)
        issue_gather(0)

        def body(jj, carry):
            t0 = start + 2 * jj
            t1 = t0 + 1

            @pl.when(jj > 0)
            def _():
                wait_write(1)

            issue_idx(t1, 1)
            wait_idx(1)
            issue_gather(1)

            wait_gather(0)
            _accum_rows(av0, bv0)
            issue_write(t0, 0)

            @pl.when(jj + 1 < niter)
            def _():
                wait_write(0)
                issue_idx(t0 + 2, 0)
                wait_idx(0)
                issue_gather(0)

            wait_gather(1)
            _accum_rows(av1, bv1)
            issue_write(t1, 1)
            return carry

        lax.fori_loop(0, niter, body, 0)
        wait_write(0)
        wait_write(1)

        for k in range(rem_rounds):
            t = per_w * NW + k * NW + wid

            @pl.when(t < nchunk)
            def _():
                issue_idx(t, 0)
                wait_idx(0)
                issue_gather(0)
                wait_gather(0)
                _accum_rows(av0, bv0)
                issue_write(t, 0)
                wait_write(0)

    return gather_kernel


_gather_half = _make_gather_add(E2)


NP = 10112
SC_CHUNKS = E2 // (NC * CH)
SC_PER_T = (SC_CHUNKS // NS) & ~1
SC_REM = SC_CHUNKS - SC_PER_T * NS
SC_ROUNDS = -(-SC_REM // NS)


@functools.partial(
    pl.kernel,
    out_type=jax.ShapeDtypeStruct((NC, NP, H), f32),
    mesh=_mesh,
    scratch_types=[
        pltpu.VMEM_SHARED((NP, H), f32),
        pltpu.VMEM((CH,), i32), pltpu.VMEM((CH,), i32),
        pltpu.VMEM((CH, H), f32), pltpu.VMEM((CH, H), f32),
        pltpu.SemaphoreType.DMA, pltpu.SemaphoreType.DMA,
        pltpu.SemaphoreType.DMA, pltpu.SemaphoreType.DMA,
    ],
)
def _scatter_half_sc(m_hbm, row_hbm, zero_hbm, out_hbm,
                     acc_sh, iv0, iv1, mv0, mv1, sl0, sl1, ss0, ss1):
    cid = lax.axis_index("c")
    sid = lax.axis_index("s")
    rows_per_tile = NP // NS
    r0 = sid * rows_per_tile
    pltpu.sync_copy(zero_hbm.at[pl.ds(r0, rows_per_tile)],
                    acc_sh.at[pl.ds(r0, rows_per_tile)])
    plsc.subcore_barrier()

    iv = (iv0, iv1)
    mv = (mv0, mv1)
    sl = (sl0, sl1)
    ss = (ss0, ss1)
    core0 = cid * SC_CHUNKS
    start = core0 + sid * SC_PER_T

    def issue_load(t, p):
        base = t * CH
        pltpu.async_copy(row_hbm.at[pl.ds(base, CH)], iv[p], sl[p])
        pltpu.async_copy(m_hbm.at[pl.ds(base, CH)], mv[p], sl[p])

    def wait_load(p):
        pltpu.make_async_copy(row_hbm.at[pl.ds(0, CH)], iv[p],
                              sl[p]).wait()
        pltpu.make_async_copy(m_hbm.at[pl.ds(0, CH)], mv[p],
                              sl[p]).wait()

    def issue_scat(p):
        pltpu.async_copy(mv[p], acc_sh.at[iv[p]], ss[p], add=True)

    def wait_scat(p):
        pltpu.make_async_copy(m_hbm.at[pl.ds(0, CH)], mv[p],
                              ss[p]).wait()

    issue_load(start, 0)

    def body(jj, carry):
        t0 = start + 2 * jj
        t1 = t0 + 1

        @pl.when(jj > 0)
        def _():
            wait_scat(1)

        issue_load(t1, 1)
        wait_load(0)
        issue_scat(0)

        @pl.when(jj + 1 < SC_PER_T // 2)
        def _():
            wait_scat(0)
            issue_load(t0 + 2, 0)

        wait_load(1)
        issue_scat(1)
        return carry

    lax.fori_loop(0, SC_PER_T // 2, body, 0)
    wait_scat(0)
    wait_scat(1)

    for k in range(SC_ROUNDS):
        t = core0 + NS * SC_PER_T + k * NS + sid

        @pl.when(t < core0 + SC_CHUNKS)
        def _():
            issue_load(t, 0)
            wait_load(0)
            issue_scat(0)
            wait_scat(0)

    plsc.subcore_barrier()
    pltpu.sync_copy(acc_sh.at[pl.ds(r0, rows_per_tile)],
                    out_hbm.at[cid, pl.ds(r0, rows_per_tile)])


BN = 1000
BE = 4000


def _node_first_body(h_ref, wemb_ref, bemb_ref, w1s_ref, w1d_ref,
                     hh_ref, a_ref, b_ref):
    hh = _bdot(h_ref[...], wemb_ref[...]) + bemb_ref[...]
    hh_ref[...] = hh
    a_ref[...] = _bdot(hh, w1s_ref[...])
    b_ref[...] = _bdot(hh, w1d_ref[...])


def _node_mid_body(hh_ref, agga_ref, aggb_ref, nw1a_ref, nw1b_ref, nb1_ref,
                   nw2_ref, nb2_ref, w1s_ref, w1d_ref,
                   hh_out_ref, a_ref, b_ref):
    hh = hh_ref[...]
    agg = (agga_ref[0] + agga_ref[1]) + (aggb_ref[0] + aggb_ref[1])
    u = _silu(_bdot(hh, nw1a_ref[...]) + _bdot(agg, nw1b_ref[...])
              + nb1_ref[...])
    out = _bdot(u, nw2_ref[...]) + nb2_ref[...]
    hh = hh + out
    hh_out_ref[...] = hh
    a_ref[...] = _bdot(hh, w1s_ref[...])
    b_ref[...] = _bdot(hh, w1d_ref[...])


def _node_last_body(hh_ref, agga_ref, aggb_ref, nw1a_ref, nw1b_ref, nb1_ref,
                    nw2_ref, nb2_ref, wout_ref, bout_ref, hout_ref):
    hh = hh_ref[...]
    agg = (agga_ref[0] + agga_ref[1]) + (aggb_ref[0] + aggb_ref[1])
    u = _silu(_bdot(hh, nw1a_ref[...]) + _bdot(agg, nw1b_ref[...])
              + nb1_ref[...])
    out = _bdot(u, nw2_ref[...]) + nb2_ref[...]
    hh = hh + out
    hout_ref[...] = _bdot(hh, wout_ref[...]) + bout_ref[...]


def _edge_body(s_ref, rad_ref, ea_ref, w1rad_ref, w1e_ref,
               b1_ref, w2_ref, b2_ref, m_ref):
    pre = (s_ref[...]
           + rad_ref[...] * w1rad_ref[...]
           + _bdot(ea_ref[...], w1e_ref[...])
           + b1_ref[...])
    t = _silu(pre)
    mm = _bdot(t, w2_ref[...]) + b2_ref[...]
    m_ref[...] = _silu(mm)


def _wspec(shape):
    return pl.BlockSpec(shape, lambda i: tuple(0 for _ in shape))


def _node_call(body, out_dtypes, extra_in_specs):
    grid = N // BN
    rowspec = pl.BlockSpec((BN, H), lambda i: (i, 0))
    in_specs = [rowspec] + extra_in_specs
    out_specs = [rowspec] * len(out_dtypes)
    out_shape = [jax.ShapeDtypeStruct((N, H), dt) for dt in out_dtypes]
    if len(out_dtypes) == 1:
        out_specs = out_specs[0]
        out_shape = out_shape[0]
    return pl.pallas_call(body, grid=grid, in_specs=in_specs,
                          out_specs=out_specs, out_shape=out_shape)


_W = _wspec((H, H))
_BIAS = _wspec((1, H))
_AGGSPEC = pl.BlockSpec((NC, BN, H), lambda i: (0, i, 0))

_node_first = _node_call(_node_first_body, [f32, f32, f32],
                         [_W, _BIAS, _W, _W])
_node_mid = _node_call(_node_mid_body, [f32, f32, f32],
                       [_AGGSPEC, _AGGSPEC, _W, _W, _BIAS, _W, _BIAS, _W, _W])
_node_last = _node_call(_node_last_body, [f32],
                        [_AGGSPEC, _AGGSPEC, _W, _W, _BIAS, _W, _BIAS, _W,
                         _BIAS])

_edge_mlp_half = pl.pallas_call(
    _edge_body,
    grid=E2 // BE,
    in_specs=[
        pl.BlockSpec((BE, H), lambda i: (i, 0)),
        pl.BlockSpec((BE, 1), lambda i: (i, 0)),
        pl.BlockSpec((BE, 16), lambda i: (i, 0)),
        _BIAS,
        _wspec((16, H)),
        _BIAS,
        _W,
        _BIAS,
    ],
    out_specs=pl.BlockSpec((BE, H), lambda i: (i, 0)),
    out_shape=jax.ShapeDtypeStruct((E2, H), f32),
)


def kernel(h, x, edge_attr, emb_in_w, emb_in_b, edge_w1, edge_b1, edge_w2,
           edge_b2, node_w1, node_b1, node_w2, node_b2, emb_out_w, emb_out_b,
           edge_index):
    L = edge_w1.shape[0]
    row = edge_index[0]
    col = edge_index[1]

    zeros_nh = jnp.zeros((NP, H), f32)

    rows = (row[:E2], row[E2:])
    cols = (col[:E2], col[E2:])
    eas = (edge_attr[:E2], edge_attr[E2:])

    w1s = [edge_w1[i, :H] for i in range(L)]
    w1d = [edge_w1[i, H:2 * H] for i in range(L)]
    w1rad = [edge_w1[i, 2 * H].reshape(1, H) for i in range(L)]
    w1e = [edge_w1[i, 2 * H + 1:] for i in range(L)]
    eb1 = [edge_b1[i].reshape(1, H) for i in range(L)]
    eb2 = [edge_b2[i].reshape(1, H) for i in range(L)]
    nw1a = [node_w1[i, :H] for i in range(L)]
    nw1b = [node_w1[i, H:] for i in range(L)]
    nb1 = [node_b1[i].reshape(1, H) for i in range(L)]
    nb2 = [node_b2[i].reshape(1, H) for i in range(L)]
    nw2 = [node_w2[i] for i in range(L)]
    ew2 = [edge_w2[i] for i in range(L)]

    hh, a, b = _node_first(h, emb_in_w, emb_in_b.reshape(1, H),
                           w1s[0], w1d[0])
    xt = jnp.zeros((4, N), f32).at[:3].set(x.T)
    radial = _radial_sc(xt, row, col).reshape(E, 1)
    rads = (radial[:E2], radial[E2:])
    h_out = None
    for i in range(L):
        m = [None, None]
        for k in range(2):
            s = _gather_half(a, b, rows[k], cols[k])
            m[k] = _edge_mlp_half(s, rads[k], eas[k], w1rad[i], w1e[i],
                                  eb1[i], ew2[i], eb2[i])
        agga = _scatter_half_sc(m[0], rows[0], zeros_nh)
        aggb = _scatter_half_sc(m[1], rows[1], zeros_nh)
        if i < L - 1:
            hh, a, b = _node_mid(hh, agga, aggb, nw1a[i], nw1b[i], nb1[i],
                                 nw2[i], nb2[i], w1s[i + 1], w1d[i + 1])
        else:
            h_out = _node_last(hh, agga, aggb, nw1a[i], nw1b[i], nb1[i],
                               nw2[i], nb2[i], emb_out_w,
                               emb_out_b.reshape(1, H))
    return (x, h_out)


# speedup vs baseline: 1.2006x; 1.0034x over previous
import functools

import jax
import jax.numpy as jnp
from jax import lax
from jax.experimental import pallas as pl
from jax.experimental.pallas import tpu as pltpu
from jax.experimental.pallas import tpu_sc as plsc

N = 10000
E = 320000
E2 = E // 2
H = 128
NC = 2
NS = 16
NW = NC * NS
CH = 128

_mesh = plsc.VectorSubcoreMesh(core_axis_name="c", subcore_axis_name="s")

f32 = jnp.float32
bf16 = jnp.bfloat16
i32 = jnp.int32


def _silu(v):
    return v * (1.0 / (1.0 + jnp.exp(-v)))


def _bdot(x, w):
    return jnp.dot(x, w, preferred_element_type=f32)


@functools.partial(
    pl.kernel,
    out_type=jax.ShapeDtypeStruct((E,), f32),
    mesh=_mesh,
    compiler_params=pltpu.CompilerParams(needs_layout_passes=False),
    scratch_types=[
        pltpu.VMEM((4, N), f32),
        pltpu.VMEM((CH,), i32),
        pltpu.VMEM((CH,), i32),
        pltpu.VMEM((CH,), f32),
    ],
)
def _radial_sc(xt_hbm, row_hbm, col_hbm, rad_hbm, xt_v, row_v, col_v, rad_v):
    cid = lax.axis_index("c")
    sid = lax.axis_index("s")
    wid = sid * NC + cid
    pltpu.sync_copy(xt_hbm, xt_v)
    nchunk = E // CH

    def body(j, carry):
        chunk = wid + j * NW

        @pl.when(chunk < nchunk)
        def _():
            base = chunk * CH
            pltpu.sync_copy(row_hbm.at[pl.ds(base, CH)], row_v)
            pltpu.sync_copy(col_hbm.at[pl.ds(base, CH)], col_v)

            def sub(k, c2):
                ridx = row_v[pl.ds(k * 16, 16)]
                cidx = col_v[pl.ds(k * 16, 16)]
                acc = jnp.zeros((16,), f32)
                for d in range(3):
                    didx = jnp.full((16,), d, i32)
                    xr = plsc.load_gather(xt_v, [didx, ridx])
                    xc = plsc.load_gather(xt_v, [didx, cidx])
                    dd = xr - xc
                    acc = acc + dd * dd
                rad_v[pl.ds(k * 16, 16)] = acc
                return c2

            lax.fori_loop(0, CH // 16, sub, 0)
            pltpu.sync_copy(rad_v, rad_hbm.at[pl.ds(base, CH)])

        return carry

    lax.fori_loop(0, (nchunk + NW - 1) // NW, body, 0)


def _accum_rows(dst_v, src_v):
    def rowbody(r, c):
        for q in range(H // 16):
            sl = pl.ds(q * 16, 16)
            dst_v[r, sl] = dst_v[r, sl] + src_v[r, sl]
        return c

    lax.fori_loop(0, CH, rowbody, 0)


def _make_gather_add(esz):
    nchunk = esz // CH
    per_w = (nchunk // NW) & ~1
    niter = per_w // 2
    rem = nchunk - per_w * NW
    rem_rounds = -(-rem // NW)

    @functools.partial(
        pl.kernel,
        out_type=jax.ShapeDtypeStruct((esz, H), f32),
        mesh=_mesh,
        scratch_types=[
            pltpu.VMEM((CH,), i32), pltpu.VMEM((CH,), i32),
            pltpu.VMEM((CH,), i32), pltpu.VMEM((CH,), i32),
            pltpu.VMEM((CH, H), f32), pltpu.VMEM((CH, H), f32),
            pltpu.VMEM((CH, H), f32), pltpu.VMEM((CH, H), f32),
            pltpu.SemaphoreType.DMA, pltpu.SemaphoreType.DMA,
            pltpu.SemaphoreType.DMA, pltpu.SemaphoreType.DMA,
            pltpu.SemaphoreType.DMA, pltpu.SemaphoreType.DMA,
        ],
    )
    def gather_kernel(a_hbm, b_hbm, row_hbm, col_hbm, s_hbm,
                      rv0, rv1, cv0, cv1, av0, av1, bv0, bv1,
                      si0, si1, sg0, sg1, sw0, sw1):
        cid = lax.axis_index("c")
        sid = lax.axis_index("s")
        wid = sid * NC + cid
        start = wid * per_w
        rv = (rv0, rv1)
        cv = (cv0, cv1)
        av = (av0, av1)
        bv = (bv0, bv1)
        si = (si0, si1)
        sg = (sg0, sg1)
        sw = (sw0, sw1)

        def issue_idx(t, p):
            base = t * CH
            pltpu.async_copy(row_hbm.at[pl.ds(base, CH)], rv[p], si[p])
            pltpu.async_copy(col_hbm.at[pl.ds(base, CH)], cv[p], si[p])

        def wait_idx(p):
            pltpu.make_async_copy(row_hbm.at[pl.ds(0, CH)], rv[p],
                                  si[p]).wait()
            pltpu.make_async_copy(col_hbm.at[pl.ds(0, CH)], cv[p],
                                  si[p]).wait()

        def issue_gather(p):
            pltpu.async_copy(a_hbm.at[rv[p]], av[p], sg[p])
            pltpu.async_copy(b_hbm.at[cv[p]], bv[p], sg[p])

        def wait_gather(p):
            pltpu.make_async_copy(a_hbm.at[pl.ds(0, CH)], av[p],
                                  sg[p]).wait()
            pltpu.make_async_copy(b_hbm.at[pl.ds(0, CH)], bv[p],
                                  sg[p]).wait()

        def issue_write(t, p):
            pltpu.async_copy(av[p], s_hbm.at[pl.ds(t * CH, CH)], sw[p])

        def wait_write(p):
            pltpu.make_async_copy(a_hbm.at[pl.ds(0, CH)], av[p],
                                  sw[p]).wait()

        issue_idx(start, 0)
        wait_idx(0)
        issue_gather(0)

        def body(jj, carry):
            t0 = start + 2 * jj
            t1 = t0 + 1

            @pl.when(jj > 0)
            def _():
                wait_write(1)

            issue_idx(t1, 1)
            wait_idx(1)
            issue_gather(1)

            wait_gather(0)
            _accum_rows(av0, bv0)
            issue_write(t0, 0)

            @pl.when(jj + 1 < niter)
            def _():
                wait_write(0)
                issue_idx(t0 + 2, 0)
                wait_idx(0)
                issue_gather(0)

            wait_gather(1)
            _accum_rows(av1, bv1)
            issue_write(t1, 1)
            return carry

        lax.fori_loop(0, niter, body, 0)
        wait_write(0)
        wait_write(1)

        for k in range(rem_rounds):
            t = per_w * NW + k * NW + wid

            @pl.when(t < nchunk)
            def _():
                issue_idx(t, 0)
                wait_idx(0)
                issue_gather(0)
                wait_gather(0)
                _accum_rows(av0, bv0)
                issue_write(t, 0)
                wait_write(0)

    return gather_kernel


_gather_half = _make_gather_add(E2)


NP = 10112
SC_CHUNKS = E2 // (NC * CH)
SC_PER_T = (SC_CHUNKS // NS) & ~1
SC_REM = SC_CHUNKS - SC_PER_T * NS
SC_ROUNDS = -(-SC_REM // NS)


@functools.partial(
    pl.kernel,
    out_type=jax.ShapeDtypeStruct((NC, NP, H), f32),
    mesh=_mesh,
    scratch_types=[
        pltpu.VMEM_SHARED((NP, H), f32),
        pltpu.VMEM((CH,), i32), pltpu.VMEM((CH,), i32),
        pltpu.VMEM((CH, H), f32), pltpu.VMEM((CH, H), f32),
        pltpu.SemaphoreType.DMA, pltpu.SemaphoreType.DMA,
        pltpu.SemaphoreType.DMA, pltpu.SemaphoreType.DMA,
    ],
)
def _scatter_half_sc(m_hbm, row_hbm, zero_hbm, out_hbm,
                     acc_sh, iv0, iv1, mv0, mv1, sl0, sl1, ss0, ss1):
    cid = lax.axis_index("c")
    sid = lax.axis_index("s")
    rows_per_tile = NP // NS
    r0 = sid * rows_per_tile
    pltpu.sync_copy(zero_hbm.at[pl.ds(r0, rows_per_tile)],
                    acc_sh.at[pl.ds(r0, rows_per_tile)])
    plsc.subcore_barrier()

    iv = (iv0, iv1)
    mv = (mv0, mv1)
    sl = (sl0, sl1)
    ss = (ss0, ss1)
    core0 = cid * SC_CHUNKS
    start = core0 + sid * SC_PER_T

    def issue_load(t, p):
        base = t * CH
        pltpu.async_copy(row_hbm.at[pl.ds(base, CH)], iv[p], sl[p])
        pltpu.async_copy(m_hbm.at[pl.ds(base, CH)], mv[p], sl[p])

    def wait_load(p):
        pltpu.make_async_copy(row_hbm.at[pl.ds(0, CH)], iv[p],
                              sl[p]).wait()
        pltpu.make_async_copy(m_hbm.at[pl.ds(0, CH)], mv[p],
                              sl[p]).wait()

    def issue_scat(p):
        pltpu.async_copy(mv[p], acc_sh.at[iv[p]], ss[p], add=True)

    def wait_scat(p):
        pltpu.make_async_copy(m_hbm.at[pl.ds(0, CH)], mv[p],
                              ss[p]).wait()

    issue_load(start, 0)

    def body(jj, carry):
        t0 = start + 2 * jj
        t1 = t0 + 1

        @pl.when(jj > 0)
        def _():
            wait_scat(1)

        issue_load(t1, 1)
        wait_load(0)
        issue_scat(0)

        @pl.when(jj + 1 < SC_PER_T // 2)
        def _():
            wait_scat(0)
            issue_load(t0 + 2, 0)

        wait_load(1)
        issue_scat(1)
        return carry

    lax.fori_loop(0, SC_PER_T // 2, body, 0)
    wait_scat(0)
    wait_scat(1)

    for k in range(SC_ROUNDS):
        t = core0 + NS * SC_PER_T + k * NS + sid

        @pl.when(t < core0 + SC_CHUNKS)
        def _():
            issue_load(t, 0)
            wait_load(0)
            issue_scat(0)
            wait_scat(0)

    plsc.subcore_barrier()
    pltpu.sync_copy(acc_sh.at[pl.ds(r0, rows_per_tile)],
                    out_hbm.at[cid, pl.ds(r0, rows_per_tile)])


BN = 1000
BE = 8000


def _node_first_body(h_ref, wemb_ref, bemb_ref, w1s_ref, w1d_ref,
                     hh_ref, a_ref, b_ref):
    hh = _bdot(h_ref[...], wemb_ref[...]) + bemb_ref[...]
    hh_ref[...] = hh
    a_ref[...] = _bdot(hh, w1s_ref[...])
    b_ref[...] = _bdot(hh, w1d_ref[...])


def _node_mid_body(hh_ref, agga_ref, aggb_ref, nw1a_ref, nw1b_ref, nb1_ref,
                   nw2_ref, nb2_ref, w1s_ref, w1d_ref,
                   hh_out_ref, a_ref, b_ref):
    hh = hh_ref[...]
    agg = (agga_ref[0] + agga_ref[1]) + (aggb_ref[0] + aggb_ref[1])
    u = _silu(_bdot(hh, nw1a_ref[...]) + _bdot(agg, nw1b_ref[...])
              + nb1_ref[...])
    out = _bdot(u, nw2_ref[...]) + nb2_ref[...]
    hh = hh + out
    hh_out_ref[...] = hh
    a_ref[...] = _bdot(hh, w1s_ref[...])
    b_ref[...] = _bdot(hh, w1d_ref[...])


def _node_last_body(hh_ref, agga_ref, aggb_ref, nw1a_ref, nw1b_ref, nb1_ref,
                    nw2_ref, nb2_ref, wout_ref, bout_ref, hout_ref):
    hh = hh_ref[...]
    agg = (agga_ref[0] + agga_ref[1]) + (aggb_ref[0] + aggb_ref[1])
    u = _silu(_bdot(hh, nw1a_ref[...]) + _bdot(agg, nw1b_ref[...])
              + nb1_ref[...])
    out = _bdot(u, nw2_ref[...]) + nb2_ref[...]
    hh = hh + out
    hout_ref[...] = _bdot(hh, wout_ref[...]) + bout_ref[...]


def _edge_body(s_ref, rad_ref, ea_ref, w1rad_ref, w1e_ref,
               b1_ref, w2_ref, b2_ref, m_ref):
    pre = (s_ref[...]
           + rad_ref[...] * w1rad_ref[...]
           + _bdot(ea_ref[...], w1e_ref[...])
           + b1_ref[...])
    t = _silu(pre)
    mm = _bdot(t, w2_ref[...]) + b2_ref[...]
    m_ref[...] = _silu(mm)


def _wspec(shape):
    return pl.BlockSpec(shape, lambda i: tuple(0 for _ in shape))


def _node_call(body, out_dtypes, extra_in_specs):
    grid = N // BN
    rowspec = pl.BlockSpec((BN, H), lambda i: (i, 0))
    in_specs = [rowspec] + extra_in_specs
    out_specs = [rowspec] * len(out_dtypes)
    out_shape = [jax.ShapeDtypeStruct((N, H), dt) for dt in out_dtypes]
    if len(out_dtypes) == 1:
        out_specs = out_specs[0]
        out_shape = out_shape[0]
    return pl.pallas_call(body, grid=grid, in_specs=in_specs,
                          out_specs=out_specs, out_shape=out_shape)


_W = _wspec((H, H))
_BIAS = _wspec((1, H))
_AGGSPEC = pl.BlockSpec((NC, BN, H), lambda i: (0, i, 0))

_node_first = _node_call(_node_first_body, [f32, f32, f32],
                         [_W, _BIAS, _W, _W])
_node_mid = _node_call(_node_mid_body, [f32, f32, f32],
                       [_AGGSPEC, _AGGSPEC, _W, _W, _BIAS, _W, _BIAS, _W, _W])
_node_last = _node_call(_node_last_body, [f32],
                        [_AGGSPEC, _AGGSPEC, _W, _W, _BIAS, _W, _BIAS, _W,
                         _BIAS])

_edge_mlp_half = pl.pallas_call(
    _edge_body,
    grid=E2 // BE,
    in_specs=[
        pl.BlockSpec((BE, H), lambda i: (i, 0)),
        pl.BlockSpec((BE, 1), lambda i: (i, 0)),
        pl.BlockSpec((BE, 16), lambda i: (i, 0)),
        _BIAS,
        _wspec((16, H)),
        _BIAS,
        _W,
        _BIAS,
    ],
    out_specs=pl.BlockSpec((BE, H), lambda i: (i, 0)),
    out_shape=jax.ShapeDtypeStruct((E2, H), f32),
)


def kernel(h, x, edge_attr, emb_in_w, emb_in_b, edge_w1, edge_b1, edge_w2,
           edge_b2, node_w1, node_b1, node_w2, node_b2, emb_out_w, emb_out_b,
           edge_index):
    L = edge_w1.shape[0]
    row = edge_index[0]
    col = edge_index[1]

    zeros_nh = jnp.zeros((NP, H), f32)

    rows = (row[:E2], row[E2:])
    cols = (col[:E2], col[E2:])
    eas = (edge_attr[:E2], edge_attr[E2:])

    w1s = [edge_w1[i, :H] for i in range(L)]
    w1d = [edge_w1[i, H:2 * H] for i in range(L)]
    w1rad = [edge_w1[i, 2 * H].reshape(1, H) for i in range(L)]
    w1e = [edge_w1[i, 2 * H + 1:] for i in range(L)]
    eb1 = [edge_b1[i].reshape(1, H) for i in range(L)]
    eb2 = [edge_b2[i].reshape(1, H) for i in range(L)]
    nw1a = [node_w1[i, :H] for i in range(L)]
    nw1b = [node_w1[i, H:] for i in range(L)]
    nb1 = [node_b1[i].reshape(1, H) for i in range(L)]
    nb2 = [node_b2[i].reshape(1, H) for i in range(L)]
    nw2 = [node_w2[i] for i in range(L)]
    ew2 = [edge_w2[i] for i in range(L)]

    hh, a, b = _node_first(h, emb_in_w, emb_in_b.reshape(1, H),
                           w1s[0], w1d[0])
    xt = jnp.zeros((4, N), f32).at[:3].set(x.T)
    radial = _radial_sc(xt, row, col).reshape(E, 1)
    rads = (radial[:E2], radial[E2:])
    h_out = None
    for i in range(L):
        m = [None, None]
        for k in range(2):
            s = _gather_half(a, b, rows[k], cols[k])
            m[k] = _edge_mlp_half(s, rads[k], eas[k], w1rad[i], w1e[i],
                                  eb1[i], ew2[i], eb2[i])
        agga = _scatter_half_sc(m[0], rows[0], zeros_nh)
        aggb = _scatter_half_sc(m[1], rows[1], zeros_nh)
        if i < L - 1:
            hh, a, b = _node_mid(hh, agga, aggb, nw1a[i], nw1b[i], nb1[i],
                                 nw2[i], nb2[i], w1s[i + 1], w1d[i + 1])
        else:
            h_out = _node_last(hh, agga, aggb, nw1a[i], nw1b[i], nb1[i],
                               nw2[i], nb2[i], emb_out_w,
                               emb_out_b.reshape(1, H))
    return (x, h_out)
